# Initial kernel scaffold; baseline (speedup 1.0000x reference)
import jax
import jax.numpy as jnp
from jax.experimental import pallas as pl


def kernel(h, phrase_mask, phrase_token_idx, W_dq, g_q, W_uq, W_kv_mh, W_z_mh, B_pos_mh, W_kv_sh, W_z_sh, B_pos_sh, W_iuq, W_w, W_k, W_v, W_o):
    raise NotImplementedError("write your pallas kernel here")


# speedup vs baseline: 3.6208x; 3.6208x over previous
import functools
import numpy as np
import jax
import jax.numpy as jnp
from jax import lax
from jax.experimental import pallas as pl
from jax.experimental.pallas import tpu as pltpu
from jax.experimental.pallas import tpu_sc as plsc

_PREC = lax.Precision.HIGHEST
_NEG_INF = np.float32(-np.inf)


def _rope_cache_np(seq_len, head_dim, base=10000.0):
    half = head_dim // 2
    inv_freq = 1.0 / (base ** (np.arange(half, dtype=np.float64) / half))
    t = np.arange(seq_len, dtype=np.float64)
    freqs = np.outer(t, inv_freq)
    emb = np.concatenate([freqs, freqs], axis=-1)
    return np.cos(emb).astype(np.float32), np.sin(emb).astype(np.float32)


def _rope_tiled(x, c, s, hd):
    lane = lax.broadcasted_iota(jnp.int32, x.shape, 1) % hd
    lo = lane < (hd // 2)
    rot = jnp.where(lo, -jnp.roll(x, -(hd // 2), axis=1), jnp.roll(x, hd // 2, axis=1))
    return x * c + rot * s


def _stage_a_body(h_ref, wcat_ref, ww_ref, wuq_ref, wiuq_ref, gq_ref,
                  cos_ref, sin_ref, cosi_ref, sini_ref,
                  kv_ref, z_ref, kraw_ref, vraw_ref, sh_ref, q_ref, qi_ref, wh_ref,
                  *, D, DQ, HC):
    x = h_ref[...]
    dn = (((1,), (0,)), ((), ()))
    hp = lax.dot_general(x, wcat_ref[...], dn, precision=_PREC,
                         preferred_element_type=jnp.float32)
    kv_ref[...] = hp[:, 0:HC]
    z_ref[...] = hp[:, HC:2 * HC]
    kpre = hp[:, 2 * HC:3 * HC]
    vraw_ref[...] = hp[:, 3 * HC:4 * HC]
    sh_ref[...] = hp[:, 4 * HC:4 * HC + 128]
    ql = hp[:, 4 * HC + 128:4 * HC + 128 + DQ]
    ms = jnp.mean(ql * ql, axis=1, keepdims=True)
    ql = ql * lax.rsqrt(ms + 1e-6) * gq_ref[...]
    wh_ref[...] = lax.dot_general(x, ww_ref[...], dn, precision=_PREC,
                                  preferred_element_type=jnp.float32)
    cos_t = cos_ref[...]
    sin_t = sin_ref[...]
    q = lax.dot_general(ql, wuq_ref[...], dn, precision=_PREC,
                        preferred_element_type=jnp.float32)
    q_ref[...] = _rope_tiled(q, cos_t, sin_t, 64)
    qi = lax.dot_general(ql, wiuq_ref[...], dn, precision=_PREC,
                         preferred_element_type=jnp.float32)
    qi_ref[...] = _rope_tiled(qi, cosi_ref[...], sini_ref[...], 64)
    kraw_ref[...] = _rope_tiled(kpre, cos_t, sin_t, 64)


def _stage_a(x, wcat, ww, wuq, wiuq, gq, cos_t, sin_t, cosi_t, sini_t):
    T, D = x.shape
    DQ = wuq.shape[0]
    HC = 1024
    TB = 128
    grid = (T // TB,)
    blk = lambda w: pl.BlockSpec((TB, w), lambda i: (i, 0))
    full = lambda a: pl.BlockSpec(a.shape, lambda i: (0, 0))
    f32 = jnp.float32
    out_shapes = [jax.ShapeDtypeStruct((T, HC), f32),
                  jax.ShapeDtypeStruct((T, HC), f32),
                  jax.ShapeDtypeStruct((T, HC), f32),
                  jax.ShapeDtypeStruct((T, HC), f32),
                  jax.ShapeDtypeStruct((T, 128), f32),
                  jax.ShapeDtypeStruct((T, HC), f32),
                  jax.ShapeDtypeStruct((T, 128), f32),
                  jax.ShapeDtypeStruct((T, 128), f32)]
    return pl.pallas_call(
        functools.partial(_stage_a_body, D=D, DQ=DQ, HC=HC),
        grid=grid,
        in_specs=[blk(D), full(wcat), full(ww), full(wuq), full(wiuq),
                  pl.BlockSpec((1, DQ), lambda i: (0, 0)),
                  blk(HC), blk(HC), blk(128), blk(128)],
        out_specs=[blk(HC), blk(HC), blk(HC), blk(HC), blk(128), blk(HC),
                   blk(128), blk(128)],
        out_shape=out_shapes,
        compiler_params=pltpu.CompilerParams(vmem_limit_bytes=60 * 1024 * 1024),
    )(x, wcat, ww, wuq, wiuq, gq, cos_t, sin_t, cosi_t, sini_t)


def _stage_b(kv, z, sh, idx_flat, idx_T, bmh, bsh, cs4,
             P, LMAX, HC, CI):
    mesh = plsc.VectorSubcoreMesh(core_axis_name="c", subcore_axis_name="s")
    NW = 32
    PW = P // NW
    NG = PW // 16
    f32 = jnp.float32
    i32 = jnp.int32

    @functools.partial(
        pl.kernel, mesh=mesh,
        out_type=[jax.ShapeDtypeStruct((P * HC,), f32),
                  jax.ShapeDtypeStruct((P * HC,), f32),
                  jax.ShapeDtypeStruct((P * CI,), f32),
                  jax.ShapeDtypeStruct((P,), i32)],
        scratch_types=[pltpu.VMEM((PW * LMAX,), i32),
                       pltpu.VMEM((LMAX * 16,), i32),
                       pltpu.VMEM((LMAX, HC), f32),
                       pltpu.VMEM((LMAX, CI), f32),
                       pltpu.VMEM((LMAX, HC), f32),
                       pltpu.VMEM((LMAX, HC), f32),
                       pltpu.VMEM((LMAX, 2 * CI), f32),
                       pltpu.VMEM((16, 256), f32),
                       pltpu.VMEM((PW * HC,), f32),
                       pltpu.VMEM((PW * HC,), f32),
                       pltpu.VMEM((PW * CI,), f32),
                       pltpu.VMEM((PW,), i32),
                       pltpu.SemaphoreType.DMA,
                       pltpu.SemaphoreType.DMA,
                       pltpu.SemaphoreType.DMA])
    def body(kv_hbm, z_hbm, sh_hbm, idx_hbm, idxT_hbm, bmh_hbm, bsh_hbm,
             cs4_hbm,
             vph_hbm, kph_hbm, kidx_hbm, ep_hbm,
             idx_v, idxT_v, bmh_v, bsh_v, crows, zrows, shrows,
             cs4r, vph_v, kph_v, kidx_v, ep_v,
             sem1, sem2, sem3):
        wid = lax.axis_index("s") * 2 + lax.axis_index("c")
        base_p = wid * PW
        pltpu.sync_copy(idx_hbm.at[pl.ds(base_p * LMAX, PW * LMAX)], idx_v)
        pltpu.sync_copy(bmh_hbm, bmh_v)
        pltpu.sync_copy(bsh_hbm, bsh_v)

        def group(g, _):
            pltpu.sync_copy(
                idxT_hbm.at[pl.ds((wid * NG + g) * 16 * LMAX, 16 * LMAX)],
                idxT_v)
            ep = idxT_v[pl.ds(0, 16)]
            for l in range(1, LMAX):
                ep = jnp.maximum(ep, idxT_v[pl.ds(l * 16, 16)])
            ep_v[pl.ds(g * 16, 16)] = ep
            pltpu.async_copy(cs4_hbm.at[ep], cs4r, sem1).wait()

            def phrase(jj, _):
                j = g * 16 + jj
                idxv = idx_v[pl.ds(j * LMAX, LMAX)]
                cp1 = pltpu.async_copy(kv_hbm.at[idxv], crows, sem1)
                cp2 = pltpu.async_copy(z_hbm.at[idxv], zrows, sem2)
                cp3 = pltpu.async_copy(sh_hbm.at[idxv], shrows, sem3)
                cp1.wait()
                cp2.wait()
                cp3.wait()

                def mh_chunk(c, _):
                    off = c * 16
                    t = [zrows[l, pl.ds(off, 16)] + bmh_v[l, pl.ds(off, 16)]
                         for l in range(LMAX)]
                    m = t[0]
                    for l in range(1, LMAX):
                        m = jnp.maximum(m, t[l])
                    se = jnp.zeros((16,), f32)
                    sec = jnp.zeros((16,), f32)
                    for l in range(LMAX):
                        e = jnp.exp(t[l] - m)
                        se = se + e
                        sec = sec + e * crows[l, pl.ds(off, 16)]
                    vph_v[pl.ds(j * HC + off, 16)] = sec / se
                    return 0

                lax.fori_loop(0, HC // 16, mh_chunk, 0)

                def sh_chunk(c, _):
                    off = c * 16
                    t = [shrows[l, pl.ds(CI + off, 16)] + bsh_v[l, pl.ds(off, 16)]
                         for l in range(LMAX)]
                    m = t[0]
                    for l in range(1, LMAX):
                        m = jnp.maximum(m, t[l])
                    se = jnp.zeros((16,), f32)
                    sec = jnp.zeros((16,), f32)
                    for l in range(LMAX):
                        e = jnp.exp(t[l] - m)
                        se = se + e
                        sec = sec + e * shrows[l, pl.ds(off, 16)]
                    kidx_v[pl.ds(j * CI + off, 16)] = sec / se
                    return 0

                lax.fori_loop(0, CI // 16, sh_chunk, 0)
                return 0

            lax.fori_loop(0, 16, phrase, 0)

            for jj in range(16):
                j = g * 16 + jj

                def kph_chunk(c, _, jj=jj, j=j):
                    off = c * 16
                    jpos = lax.rem(off, 64)
                    is_lo = jpos < 32
                    src = j * HC + (off - jpos) + jnp.where(is_lo, jpos + 32,
                                                            jpos - 32)
                    v = vph_v[pl.ds(j * HC + off, 16)]
                    rv = vph_v[pl.ds(src, 16)]
                    sgn = jnp.where(is_lo, jnp.float32(-1.0), jnp.float32(1.0))
                    cv = cs4r[jj, pl.ds(jpos, 16)]
                    sv = cs4r[jj, pl.ds(64 + jpos, 16)]
                    kph_v[pl.ds(j * HC + off, 16)] = v * cv + sgn * rv * sv
                    return 0

                lax.fori_loop(0, HC // 16, kph_chunk, 0)

                r0 = kidx_v[pl.ds(j * CI + 0, 16)]
                r1 = kidx_v[pl.ds(j * CI + 16, 16)]
                r2 = kidx_v[pl.ds(j * CI + 32, 16)]
                r3 = kidx_v[pl.ds(j * CI + 48, 16)]
                c0 = cs4r[jj, pl.ds(128, 16)]
                c1 = cs4r[jj, pl.ds(144, 16)]
                c2 = cs4r[jj, pl.ds(160, 16)]
                c3 = cs4r[jj, pl.ds(176, 16)]
                s0 = cs4r[jj, pl.ds(192, 16)]
                s1 = cs4r[jj, pl.ds(208, 16)]
                s2 = cs4r[jj, pl.ds(224, 16)]
                s3 = cs4r[jj, pl.ds(240, 16)]
                kidx_v[pl.ds(j * CI + 0, 16)] = r0 * c0 - r2 * s0
                kidx_v[pl.ds(j * CI + 16, 16)] = r1 * c1 - r3 * s1
                kidx_v[pl.ds(j * CI + 32, 16)] = r2 * c2 + r0 * s2
                kidx_v[pl.ds(j * CI + 48, 16)] = r3 * c3 + r1 * s3
            return 0

        lax.fori_loop(0, NG, group, 0)
        pltpu.sync_copy(vph_v, vph_hbm.at[pl.ds(base_p * HC, PW * HC)])
        pltpu.sync_copy(kph_v, kph_hbm.at[pl.ds(base_p * HC, PW * HC)])
        pltpu.sync_copy(kidx_v, kidx_hbm.at[pl.ds(base_p * CI, PW * CI)])
        pltpu.sync_copy(ep_v, ep_hbm.at[pl.ds(base_p, PW)])

    return body(kv, z, sh, idx_flat, idx_T, bmh, bsh, cs4)


def _stage_c_body(qi_ref, kidx_ref, wh_ref, ep_ref, sel_ref, *, TB, P, TOPK):
    i = pl.program_id(0)
    qi = qi_ref[...]
    dn = (((1,), (1,)), ((), ()))
    s0 = lax.dot_general(qi[:, :64], kidx_ref[...], dn, precision=_PREC,
                         preferred_element_type=jnp.float32)
    s1 = lax.dot_general(qi[:, 64:128], kidx_ref[...], dn, precision=_PREC,
                         preferred_element_type=jnp.float32)
    wh = wh_ref[...]
    sc = jnp.maximum(s0, 0.0) * wh[:, 0:1] + jnp.maximum(s1, 0.0) * wh[:, 1:2]
    t = i * TB + lax.broadcasted_iota(jnp.int32, (TB, 1), 0)
    causal = ep_ref[...] <= t
    sc = jnp.where(causal, sc, _NEG_INF)
    iota_p = lax.broadcasted_iota(jnp.int32, (TB, P), 1)

    def it(_, carry):
        sc, sel = carry
        mval = jnp.max(sc, axis=1, keepdims=True)
        amax = jnp.min(jnp.where(sc == mval, iota_p, P), axis=1, keepdims=True)
        pick = iota_p == amax
        valid = mval > _NEG_INF
        sel = jnp.where(pick & valid, 1.0, sel)
        sc = jnp.where(pick, _NEG_INF, sc)
        return sc, sel

    _, sel = lax.fori_loop(0, TOPK, it, (sc, jnp.zeros((TB, P), jnp.float32)))
    sel_ref[...] = sel


def _stage_c(qi, kidx, wh, ep_row, TOPK):
    T = qi.shape[0]
    P = kidx.shape[0]
    TB = 128
    return pl.pallas_call(
        functools.partial(_stage_c_body, TB=TB, P=P, TOPK=TOPK),
        grid=(T // TB,),
        in_specs=[pl.BlockSpec((TB, 128), lambda i: (i, 0)),
                  pl.BlockSpec(kidx.shape, lambda i: (0, 0)),
                  pl.BlockSpec((TB, 128), lambda i: (i, 0)),
                  pl.BlockSpec((1, P), lambda i: (0, 0))],
        out_specs=pl.BlockSpec((TB, P), lambda i: (i, 0)),
        out_shape=jax.ShapeDtypeStruct((T, P), jnp.float32),
    )(qi, kidx, wh, ep_row)


def _stage_d_body(q_ref, kc_ref, kp_ref, vc_ref, vp_ref, kph_ref, vph_ref,
                  sel_ref, wo_ref, out_ref, *, TB, P, H, C):
    i = pl.program_id(0)
    scale = 1.0 / np.sqrt(np.float32(C))
    row = lax.broadcasted_iota(jnp.int32, (TB, TB), 0)
    col = lax.broadcasted_iota(jnp.int32, (TB, TB), 1)
    mask_c = col <= row
    mask_p = (row < col) & (i > 0)
    selm = sel_ref[...] > 0.0
    dnT = (((1,), (1,)), ((), ()))
    dnN = (((1,), (0,)), ((), ()))
    heads = []
    for h in range(H):
        sl = slice(h * C, (h + 1) * C)
        qh = q_ref[:, sl] * scale
        lc = lax.dot_general(qh, kc_ref[:, sl], dnT, precision=_PREC,
                             preferred_element_type=jnp.float32)
        lp = lax.dot_general(qh, kp_ref[:, sl], dnT, precision=_PREC,
                             preferred_element_type=jnp.float32)
        lph = lax.dot_general(qh, kph_ref[:, sl], dnT, precision=_PREC,
                              preferred_element_type=jnp.float32)
        lc = jnp.where(mask_c, lc, _NEG_INF)
        lp = jnp.where(mask_p, lp, _NEG_INF)
        lph = jnp.where(selm, lph, _NEG_INF)
        m = jnp.maximum(jnp.max(lc, axis=1, keepdims=True),
                        jnp.maximum(jnp.max(lp, axis=1, keepdims=True),
                                    jnp.max(lph, axis=1, keepdims=True)))
        ec = jnp.exp(lc - m)
        ep = jnp.exp(lp - m)
        eph = jnp.exp(lph - m)
        den = (jnp.sum(ec, axis=1, keepdims=True)
               + jnp.sum(ep, axis=1, keepdims=True)
               + jnp.sum(eph, axis=1, keepdims=True))
        oh = (lax.dot_general(ec, vc_ref[:, sl], dnN, precision=_PREC,
                              preferred_element_type=jnp.float32)
              + lax.dot_general(ep, vp_ref[:, sl], dnN, precision=_PREC,
                                preferred_element_type=jnp.float32)
              + lax.dot_general(eph, vph_ref[:, sl], dnN, precision=_PREC,
                                preferred_element_type=jnp.float32))
        heads.append(oh / den)
    att = jnp.concatenate(heads, axis=1)
    out_ref[...] = lax.dot_general(att, wo_ref[...], dnN, precision=_PREC,
                                   preferred_element_type=jnp.float32)


def _stage_d(q, k_raw, v_raw, kph, vph, sel, wo, H, C):
    T, HC = q.shape
    P = kph.shape[0]
    TB = 128
    cur = pl.BlockSpec((TB, HC), lambda i: (i, 0))
    prev = pl.BlockSpec((TB, HC), lambda i: (jnp.maximum(i - 1, 0), 0))
    return pl.pallas_call(
        functools.partial(_stage_d_body, TB=TB, P=P, H=H, C=C),
        grid=(T // TB,),
        in_specs=[cur, cur, prev, cur, prev,
                  pl.BlockSpec(kph.shape, lambda i: (0, 0)),
                  pl.BlockSpec(vph.shape, lambda i: (0, 0)),
                  pl.BlockSpec((TB, P), lambda i: (i, 0)),
                  pl.BlockSpec(wo.shape, lambda i: (0, 0))],
        out_specs=pl.BlockSpec((TB, HC), lambda i: (i, 0)),
        out_shape=jax.ShapeDtypeStruct((T, HC), jnp.float32),
        compiler_params=pltpu.CompilerParams(vmem_limit_bytes=60 * 1024 * 1024),
    )(q, k_raw, k_raw, v_raw, v_raw, kph, vph, sel, wo)


def kernel(h, phrase_mask, phrase_token_idx, W_dq, g_q, W_uq, W_kv_mh, W_z_mh,
           B_pos_mh, W_kv_sh, W_z_sh, B_pos_sh, W_iuq, W_w, W_k, W_v, W_o):
    Bb, T, D = h.shape
    DQ = W_dq.shape[1]
    HC = W_uq.shape[1]
    C = 64
    H = HC // C
    CI = W_kv_sh.shape[1]
    P, LMAX = phrase_mask.shape[1], phrase_mask.shape[2]
    TOPK = 32

    cos_np, sin_np = _rope_cache_np(T, C)
    cosi_np, sini_np = _rope_cache_np(T, CI)
    cos_t = jnp.asarray(np.tile(cos_np, (1, H)))
    sin_t = jnp.asarray(np.tile(sin_np, (1, H)))
    cosi_t = jnp.asarray(np.tile(cosi_np, (1, 2)))
    sini_t = jnp.asarray(np.tile(sini_np, (1, 2)))
    cs4 = jnp.asarray(np.concatenate([cos_np, sin_np, cosi_np, sini_np], axis=1))

    x = h[0]
    wcat = jnp.concatenate([W_kv_mh, W_z_mh, W_k, W_v, W_kv_sh, W_z_sh, W_dq],
                           axis=1)
    ww = jnp.pad(W_w, ((0, 0), (0, 128 - W_w.shape[1])))
    gq = g_q.reshape(1, DQ)

    kv, z, k_raw, v_raw, sh, q, qi, wh = _stage_a(
        x, wcat, ww, wuq := W_uq, wiuq := W_iuq, gq, cos_t, sin_t, cosi_t, sini_t)

    idx2 = phrase_token_idx.reshape(P, LMAX).astype(jnp.int32)
    idx_flat = idx2.reshape(-1)
    idx_T = idx2.reshape(P // 16, 16, LMAX).transpose(0, 2, 1).reshape(-1)
    bmh = B_pos_mh.reshape(LMAX, HC)
    vph_f, kph_f, kidx_f, ep_f = _stage_b(
        kv, z, sh, idx_flat, idx_T, bmh, B_pos_sh, cs4, P, LMAX, HC, CI)
    vph = vph_f.reshape(P, HC)
    kph = kph_f.reshape(P, HC)
    kidx = kidx_f.reshape(P, CI)
    ep_row = ep_f.reshape(1, P)

    sel = _stage_c(qi, kidx, wh, ep_row, TOPK)
    out = _stage_d(q, k_raw, v_raw, kph, vph, sel, W_o, H, C)
    return out.reshape(Bb, T, HC)


# speedup vs baseline: 5.2300x; 1.4444x over previous
import functools
import numpy as np
import jax
import jax.numpy as jnp
from jax import lax
from jax.experimental import pallas as pl
from jax.experimental.pallas import tpu as pltpu
from jax.experimental.pallas import tpu_sc as plsc

_PREC = lax.Precision.HIGHEST
_PREC_FAST = lax.Precision.DEFAULT
_NEG_INF = np.float32(-np.inf)


def _rope_cache_np(seq_len, head_dim, base=10000.0):
    half = head_dim // 2
    inv_freq = 1.0 / (base ** (np.arange(half, dtype=np.float64) / half))
    t = np.arange(seq_len, dtype=np.float64)
    freqs = np.outer(t, inv_freq)
    emb = np.concatenate([freqs, freqs], axis=-1)
    return np.cos(emb).astype(np.float32), np.sin(emb).astype(np.float32)


def _rope_tiled(x, c, s, hd):
    lane = lax.broadcasted_iota(jnp.int32, x.shape, 1) % hd
    lo = lane < (hd // 2)
    rot = jnp.where(lo, -jnp.roll(x, -(hd // 2), axis=1), jnp.roll(x, hd // 2, axis=1))
    return x * c + rot * s


def _stage_a_body(h_ref, wcat_ref, wtail_ref, ww_ref, wuq_ref, wiuq_ref, gq_ref,
                  cos_ref, sin_ref, cosi_ref, sini_ref,
                  kv_ref, z_ref, kraw_ref, vraw_ref, sh_ref, q_ref, qi_ref, wh_ref,
                  *, D, DQ, HC):
    x = h_ref[...]
    dn = (((1,), (0,)), ((), ()))
    hp = lax.dot_general(x, wcat_ref[...], dn, precision=_PREC_FAST,
                         preferred_element_type=jnp.float32)
    ht = lax.dot_general(x, wtail_ref[...], dn, precision=_PREC,
                         preferred_element_type=jnp.float32)
    kv_ref[...] = hp[:, 0:HC]
    z_ref[...] = hp[:, HC:2 * HC]
    kpre = hp[:, 2 * HC:3 * HC]
    vraw_ref[...] = hp[:, 3 * HC:4 * HC]
    sh_ref[...] = ht[:, 0:128]
    ql = ht[:, 128:128 + DQ]
    ms = jnp.mean(ql * ql, axis=1, keepdims=True)
    ql = ql * lax.rsqrt(ms + 1e-6) * gq_ref[...]
    wh_ref[...] = lax.dot_general(x, ww_ref[...], dn, precision=_PREC,
                                  preferred_element_type=jnp.float32)
    cos_t = cos_ref[...]
    sin_t = sin_ref[...]
    q = lax.dot_general(ql, wuq_ref[...], dn, precision=_PREC,
                        preferred_element_type=jnp.float32)
    q_ref[...] = _rope_tiled(q, cos_t, sin_t, 64)
    qi = lax.dot_general(ql, wiuq_ref[...], dn, precision=_PREC,
                         preferred_element_type=jnp.float32)
    qi_ref[...] = _rope_tiled(qi, cosi_ref[...], sini_ref[...], 64)
    kraw_ref[...] = _rope_tiled(kpre, cos_t, sin_t, 64)


def _stage_a(x, wcat, ww, wuq, wiuq, gq, cos_t, sin_t, cosi_t, sini_t):
    T, D = x.shape
    DQ = wuq.shape[0]
    HC = 1024
    TB = 128
    grid = (T // TB,)
    wtail = wcat[:, 4 * HC:]
    wcat = wcat[:, :4 * HC]
    blk = lambda w: pl.BlockSpec((TB, w), lambda i: (i, 0))
    full = lambda a: pl.BlockSpec(a.shape, lambda i: (0, 0))
    f32 = jnp.float32
    out_shapes = [jax.ShapeDtypeStruct((T, HC), f32),
                  jax.ShapeDtypeStruct((T, HC), f32),
                  jax.ShapeDtypeStruct((T, HC), f32),
                  jax.ShapeDtypeStruct((T, HC), f32),
                  jax.ShapeDtypeStruct((T, 128), f32),
                  jax.ShapeDtypeStruct((T, HC), f32),
                  jax.ShapeDtypeStruct((T, 128), f32),
                  jax.ShapeDtypeStruct((T, 128), f32)]
    return pl.pallas_call(
        functools.partial(_stage_a_body, D=D, DQ=DQ, HC=HC),
        grid=grid,
        in_specs=[blk(D), full(wcat), full(wtail), full(ww), full(wuq),
                  full(wiuq),
                  pl.BlockSpec((1, DQ), lambda i: (0, 0)),
                  blk(HC), blk(HC), blk(128), blk(128)],
        out_specs=[blk(HC), blk(HC), blk(HC), blk(HC), blk(128), blk(HC),
                   blk(128), blk(128)],
        out_shape=out_shapes,
        compiler_params=pltpu.CompilerParams(vmem_limit_bytes=60 * 1024 * 1024),
    )(x, wcat, wtail, ww, wuq, wiuq, gq, cos_t, sin_t, cosi_t, sini_t)


def _stage_b(kv, z, sh, idx_flat, idx_T, bmh, bsh, cs4,
             P, LMAX, HC, CI):
    mesh = plsc.VectorSubcoreMesh(core_axis_name="c", subcore_axis_name="s")
    NW = 32
    PW = P // NW
    NG = PW // 16
    f32 = jnp.float32
    i32 = jnp.int32

    @functools.partial(
        pl.kernel, mesh=mesh,
        out_type=[jax.ShapeDtypeStruct((P * HC,), f32),
                  jax.ShapeDtypeStruct((P * CI,), f32),
                  jax.ShapeDtypeStruct((P,), i32),
                  jax.ShapeDtypeStruct((P, 256), f32)],
        scratch_types=[pltpu.VMEM((PW * LMAX,), i32),
                       pltpu.VMEM((LMAX * 16,), i32),
                       pltpu.VMEM((LMAX, HC), f32),
                       pltpu.VMEM((LMAX, CI), f32),
                       pltpu.VMEM((LMAX, HC), f32),
                       pltpu.VMEM((LMAX, HC), f32),
                       pltpu.VMEM((LMAX, 2 * CI), f32),
                       pltpu.VMEM((LMAX, HC), f32),
                       pltpu.VMEM((LMAX, HC), f32),
                       pltpu.VMEM((LMAX, 2 * CI), f32),
                       pltpu.VMEM((16, 256), f32),
                       pltpu.VMEM((PW * HC,), f32),
                       pltpu.VMEM((PW * CI,), f32),
                       pltpu.VMEM((PW,), i32),
                       pltpu.SemaphoreType.DMA,
                       pltpu.SemaphoreType.DMA,
                       pltpu.SemaphoreType.DMA,
                       pltpu.SemaphoreType.DMA,
                       pltpu.SemaphoreType.DMA,
                       pltpu.SemaphoreType.DMA])
    def body(kv_hbm, z_hbm, sh_hbm, idx_hbm, idxT_hbm, bmh_hbm, bsh_hbm,
             cs4_hbm,
             vph_hbm, kidx_hbm, ep_hbm, cs4pe_hbm,
             idx_v, idxT_v, bmh_v, bsh_v,
             crowsA, zrowsA, shrowsA, crowsB, zrowsB, shrowsB,
             cs4r, vph_v, kidx_v, ep_v,
             semA1, semA2, semA3, semB1, semB2, semB3):
        wid = lax.axis_index("s") * 2 + lax.axis_index("c")
        base_p = wid * PW
        pltpu.sync_copy(idx_hbm.at[pl.ds(base_p * LMAX, PW * LMAX)], idx_v)
        pltpu.sync_copy(bmh_hbm, bmh_v)
        pltpu.sync_copy(bsh_hbm, bsh_v)

        def group(g, _):
            pltpu.sync_copy(
                idxT_hbm.at[pl.ds((wid * NG + g) * 16 * LMAX, 16 * LMAX)],
                idxT_v)
            ep = idxT_v[pl.ds(0, 16)]
            for l in range(1, LMAX):
                ep = jnp.maximum(ep, idxT_v[pl.ds(l * 16, 16)])
            ep_v[pl.ds(g * 16, 16)] = ep
            pltpu.async_copy(cs4_hbm.at[ep], cs4r, semA1).wait()
            pltpu.sync_copy(cs4r, cs4pe_hbm.at[pl.ds(base_p + g * 16, 16)])
            return 0

        lax.fori_loop(0, NG, group, 0)

        bufs = ((crowsA, zrowsA, shrowsA, semA1, semA2, semA3),
                (crowsB, zrowsB, shrowsB, semB1, semB2, semB3))

        def issue(p, s):
            crows, zrows, shrows, s1, s2, s3 = bufs[s]
            idxv = idx_v[pl.ds(p * LMAX, LMAX)]
            pltpu.async_copy(kv_hbm.at[idxv], crows, s1)
            pltpu.async_copy(z_hbm.at[idxv], zrows, s2)
            pltpu.async_copy(sh_hbm.at[idxv], shrows, s3)

        def wait(s):
            crows, zrows, shrows, s1, s2, s3 = bufs[s]
            pltpu.make_async_copy(kv_hbm.at[pl.ds(0, LMAX)], crows, s1).wait()
            pltpu.make_async_copy(z_hbm.at[pl.ds(0, LMAX)], zrows, s2).wait()
            pltpu.make_async_copy(sh_hbm.at[pl.ds(0, LMAX)], shrows, s3).wait()

        def gate(p, s):
            crows, zrows, shrows = bufs[s][:3]

            def mh_chunk(c, _):
                off = c * 16
                t = [zrows[l, pl.ds(off, 16)] + bmh_v[l, pl.ds(off, 16)]
                     for l in range(LMAX)]
                m = t[0]
                for l in range(1, LMAX):
                    m = jnp.maximum(m, t[l])
                se = jnp.zeros((16,), f32)
                sec = jnp.zeros((16,), f32)
                for l in range(LMAX):
                    e = jnp.exp(t[l] - m)
                    se = se + e
                    sec = sec + e * crows[l, pl.ds(off, 16)]
                vph_v[pl.ds(p * HC + off, 16)] = sec / se
                return 0

            lax.fori_loop(0, HC // 16, mh_chunk, 0)

            def sh_chunk(c, _):
                off = c * 16
                t = [shrows[l, pl.ds(CI + off, 16)] + bsh_v[l, pl.ds(off, 16)]
                     for l in range(LMAX)]
                m = t[0]
                for l in range(1, LMAX):
                    m = jnp.maximum(m, t[l])
                se = jnp.zeros((16,), f32)
                sec = jnp.zeros((16,), f32)
                for l in range(LMAX):
                    e = jnp.exp(t[l] - m)
                    se = se + e
                    sec = sec + e * shrows[l, pl.ds(off, 16)]
                kidx_v[pl.ds(p * CI + off, 16)] = sec / se
                return 0

            lax.fori_loop(0, CI // 16, sh_chunk, 0)

        issue(0, 0)

        def pair(jj2, _):
            p0 = jj2 * 2
            issue(p0 + 1, 1)
            wait(0)
            gate(p0, 0)

            @pl.when(p0 + 2 < PW)
            def _():
                issue(p0 + 2, 0)

            wait(1)
            gate(p0 + 1, 1)
            return 0

        lax.fori_loop(0, PW // 2, pair, 0)
        pltpu.sync_copy(vph_v, vph_hbm.at[pl.ds(base_p * HC, PW * HC)])
        pltpu.sync_copy(kidx_v, kidx_hbm.at[pl.ds(base_p * CI, PW * CI)])
        pltpu.sync_copy(ep_v, ep_hbm.at[pl.ds(base_p, PW)])

    return body(kv, z, sh, idx_flat, idx_T, bmh, bsh, cs4)


def _stage_b2_body(vph_ref, kidxp_ref, cs4pe_ref, kph_ref, kidx_ref, *, H):
    cs = cs4pe_ref[...]
    cos_t = jnp.concatenate([cs[:, 0:64]] * H, axis=1)
    sin_t = jnp.concatenate([cs[:, 64:128]] * H, axis=1)
    kph_ref[...] = _rope_tiled(vph_ref[...], cos_t, sin_t, 64)
    kidx_ref[...] = _rope_tiled(kidxp_ref[...], cs[:, 128:192], cs[:, 192:256], 64)


def _stage_b2(vph, kidxp, cs4pe, H):
    P, HC = vph.shape
    CI = kidxp.shape[1]
    PB = 256
    blk = lambda w: pl.BlockSpec((PB, w), lambda i: (i, 0))
    return pl.pallas_call(
        functools.partial(_stage_b2_body, H=H),
        grid=(P // PB,),
        in_specs=[blk(HC), blk(CI), blk(256)],
        out_specs=[blk(HC), blk(CI)],
        out_shape=[jax.ShapeDtypeStruct((P, HC), jnp.float32),
                   jax.ShapeDtypeStruct((P, CI), jnp.float32)],
    )(vph, kidxp, cs4pe)


def _stage_c_body(qi_ref, kidx_ref, wh_ref, ep_ref, sel_ref, *, TB, P, TOPK):
    i = pl.program_id(0)
    qi = qi_ref[...]
    dn = (((1,), (1,)), ((), ()))
    s0 = lax.dot_general(qi[:, :64], kidx_ref[...], dn, precision=_PREC,
                         preferred_element_type=jnp.float32)
    s1 = lax.dot_general(qi[:, 64:128], kidx_ref[...], dn, precision=_PREC,
                         preferred_element_type=jnp.float32)
    wh = wh_ref[...]
    sc = jnp.maximum(s0, 0.0) * wh[:, 0:1] + jnp.maximum(s1, 0.0) * wh[:, 1:2]
    t = i * TB + lax.broadcasted_iota(jnp.int32, (TB, 1), 0)
    causal = ep_ref[...] <= t
    sc = jnp.where(causal, sc, _NEG_INF)
    iota_p = lax.broadcasted_iota(jnp.int32, (TB, P), 1)

    def it(_, carry):
        sc, sel = carry
        mval = jnp.max(sc, axis=1, keepdims=True)
        amax = jnp.min(jnp.where(sc == mval, iota_p, P), axis=1, keepdims=True)
        pick = iota_p == amax
        valid = mval > _NEG_INF
        sel = jnp.where(pick & valid, 1.0, sel)
        sc = jnp.where(pick, _NEG_INF, sc)
        return sc, sel

    _, sel = lax.fori_loop(0, TOPK, it, (sc, jnp.zeros((TB, P), jnp.float32)))
    sel_ref[...] = sel


def _stage_c(qi, kidx, wh, ep_row, TOPK):
    T = qi.shape[0]
    P = kidx.shape[0]
    TB = 128
    return pl.pallas_call(
        functools.partial(_stage_c_body, TB=TB, P=P, TOPK=TOPK),
        grid=(T // TB,),
        in_specs=[pl.BlockSpec((TB, 128), lambda i: (i, 0)),
                  pl.BlockSpec(kidx.shape, lambda i: (0, 0)),
                  pl.BlockSpec((TB, 128), lambda i: (i, 0)),
                  pl.BlockSpec((1, P), lambda i: (0, 0))],
        out_specs=pl.BlockSpec((TB, P), lambda i: (i, 0)),
        out_shape=jax.ShapeDtypeStruct((T, P), jnp.float32),
    )(qi, kidx, wh, ep_row)


def _stage_d_body(q_ref, kc_ref, kp_ref, vc_ref, vp_ref, kph_ref, vph_ref,
                  sel_ref, wo_ref, out_ref, *, TB, P, H, C):
    i = pl.program_id(0)
    scale = 1.0 / np.sqrt(np.float32(C))
    row = lax.broadcasted_iota(jnp.int32, (TB, TB), 0)
    col = lax.broadcasted_iota(jnp.int32, (TB, TB), 1)
    mask_c = col <= row
    mask_p = (row < col) & (i > 0)
    selm = sel_ref[...] > 0.0
    dnT = (((1,), (1,)), ((), ()))
    dnN = (((1,), (0,)), ((), ()))
    heads = []
    for h in range(H):
        sl = slice(h * C, (h + 1) * C)
        qh = q_ref[:, sl] * scale
        lc = lax.dot_general(qh, kc_ref[:, sl], dnT, precision=_PREC,
                             preferred_element_type=jnp.float32)
        lp = lax.dot_general(qh, kp_ref[:, sl], dnT, precision=_PREC,
                             preferred_element_type=jnp.float32)
        lph = lax.dot_general(qh, kph_ref[:, sl], dnT, precision=_PREC,
                              preferred_element_type=jnp.float32)
        lc = jnp.where(mask_c, lc, _NEG_INF)
        lp = jnp.where(mask_p, lp, _NEG_INF)
        lph = jnp.where(selm, lph, _NEG_INF)
        m = jnp.maximum(jnp.max(lc, axis=1, keepdims=True),
                        jnp.maximum(jnp.max(lp, axis=1, keepdims=True),
                                    jnp.max(lph, axis=1, keepdims=True)))
        ec = jnp.exp(lc - m)
        ep = jnp.exp(lp - m)
        eph = jnp.exp(lph - m)
        den = (jnp.sum(ec, axis=1, keepdims=True)
               + jnp.sum(ep, axis=1, keepdims=True)
               + jnp.sum(eph, axis=1, keepdims=True))
        oh = (lax.dot_general(ec, vc_ref[:, sl], dnN, precision=_PREC_FAST,
                              preferred_element_type=jnp.float32)
              + lax.dot_general(ep, vp_ref[:, sl], dnN, precision=_PREC_FAST,
                                preferred_element_type=jnp.float32)
              + lax.dot_general(eph, vph_ref[:, sl], dnN, precision=_PREC_FAST,
                                preferred_element_type=jnp.float32))
        heads.append(oh / den)
    att = jnp.concatenate(heads, axis=1)
    out_ref[...] = lax.dot_general(att, wo_ref[...], dnN, precision=_PREC_FAST,
                                   preferred_element_type=jnp.float32)


def _stage_d(q, k_raw, v_raw, kph, vph, sel, wo, H, C):
    T, HC = q.shape
    P = kph.shape[0]
    TB = 128
    cur = pl.BlockSpec((TB, HC), lambda i: (i, 0))
    prev = pl.BlockSpec((TB, HC), lambda i: (jnp.maximum(i - 1, 0), 0))
    return pl.pallas_call(
        functools.partial(_stage_d_body, TB=TB, P=P, H=H, C=C),
        grid=(T // TB,),
        in_specs=[cur, cur, prev, cur, prev,
                  pl.BlockSpec(kph.shape, lambda i: (0, 0)),
                  pl.BlockSpec(vph.shape, lambda i: (0, 0)),
                  pl.BlockSpec((TB, P), lambda i: (i, 0)),
                  pl.BlockSpec(wo.shape, lambda i: (0, 0))],
        out_specs=pl.BlockSpec((TB, HC), lambda i: (i, 0)),
        out_shape=jax.ShapeDtypeStruct((T, HC), jnp.float32),
        compiler_params=pltpu.CompilerParams(vmem_limit_bytes=60 * 1024 * 1024),
    )(q, k_raw, k_raw, v_raw, v_raw, kph, vph, sel, wo)


def kernel(h, phrase_mask, phrase_token_idx, W_dq, g_q, W_uq, W_kv_mh, W_z_mh,
           B_pos_mh, W_kv_sh, W_z_sh, B_pos_sh, W_iuq, W_w, W_k, W_v, W_o):
    Bb, T, D = h.shape
    DQ = W_dq.shape[1]
    HC = W_uq.shape[1]
    C = 64
    H = HC // C
    CI = W_kv_sh.shape[1]
    P, LMAX = phrase_mask.shape[1], phrase_mask.shape[2]
    TOPK = 32

    cos_np, sin_np = _rope_cache_np(T, C)
    cosi_np, sini_np = _rope_cache_np(T, CI)
    cos_t = jnp.asarray(np.tile(cos_np, (1, H)))
    sin_t = jnp.asarray(np.tile(sin_np, (1, H)))
    cosi_t = jnp.asarray(np.tile(cosi_np, (1, 2)))
    sini_t = jnp.asarray(np.tile(sini_np, (1, 2)))
    cs4 = jnp.asarray(np.concatenate([cos_np, sin_np, cosi_np, sini_np], axis=1))

    x = h[0]
    wcat = jnp.concatenate([W_kv_mh, W_z_mh, W_k, W_v, W_kv_sh, W_z_sh, W_dq],
                           axis=1)
    ww = jnp.pad(W_w, ((0, 0), (0, 128 - W_w.shape[1])))
    gq = g_q.reshape(1, DQ)

    kv, z, k_raw, v_raw, sh, q, qi, wh = _stage_a(
        x, wcat, ww, wuq := W_uq, wiuq := W_iuq, gq, cos_t, sin_t, cosi_t, sini_t)

    idx2 = phrase_token_idx.reshape(P, LMAX).astype(jnp.int32)
    idx_flat = idx2.reshape(-1)
    idx_T = idx2.reshape(P // 16, 16, LMAX).transpose(0, 2, 1).reshape(-1)
    bmh = B_pos_mh.reshape(LMAX, HC)
    vph_f, kidxp_f, ep_f, cs4pe = _stage_b(
        kv, z, sh, idx_flat, idx_T, bmh, B_pos_sh, cs4, P, LMAX, HC, CI)
    vph = vph_f.reshape(P, HC)
    kph, kidx = _stage_b2(vph, kidxp_f.reshape(P, CI), cs4pe, H)
    ep_row = ep_f.reshape(1, P)

    sel = _stage_c(qi, kidx, wh, ep_row, TOPK)
    out = _stage_d(q, k_raw, v_raw, kph, vph, sel, W_o, H, C)
    return out.reshape(Bb, T, HC)


# speedup vs baseline: 5.4022x; 1.0329x over previous
import functools
import numpy as np
import jax
import jax.numpy as jnp
from jax import lax
from jax.experimental import pallas as pl
from jax.experimental.pallas import tpu as pltpu
from jax.experimental.pallas import tpu_sc as plsc

_PREC = lax.Precision.HIGHEST
_PREC_FAST = lax.Precision.DEFAULT
_NEG_INF = np.float32(-np.inf)


def _rope_cache_np(seq_len, head_dim, base=10000.0):
    half = head_dim // 2
    inv_freq = 1.0 / (base ** (np.arange(half, dtype=np.float64) / half))
    t = np.arange(seq_len, dtype=np.float64)
    freqs = np.outer(t, inv_freq)
    emb = np.concatenate([freqs, freqs], axis=-1)
    return np.cos(emb).astype(np.float32), np.sin(emb).astype(np.float32)


def _rope_tiled(x, c, s, hd):
    lane = lax.broadcasted_iota(jnp.int32, x.shape, 1) % hd
    lo = lane < (hd // 2)
    rot = jnp.where(lo, -jnp.roll(x, -(hd // 2), axis=1), jnp.roll(x, hd // 2, axis=1))
    return x * c + rot * s


def _stage_a_body(h_ref, wcat_ref, wtail_ref, ww_ref, wuq_ref, wiuq_ref, gq_ref,
                  cos_ref, sin_ref, cosi_ref, sini_ref,
                  kv_ref, z_ref, kraw_ref, vraw_ref, sh_ref, q_ref, qi_ref, wh_ref,
                  *, D, DQ, HC):
    x = h_ref[...]
    dn = (((1,), (0,)), ((), ()))
    hp = lax.dot_general(x, wcat_ref[...], dn, precision=_PREC_FAST,
                         preferred_element_type=jnp.float32)
    ht = lax.dot_general(x, wtail_ref[...], dn, precision=_PREC,
                         preferred_element_type=jnp.float32)
    kv_ref[...] = hp[:, 0:HC]
    z_ref[...] = hp[:, HC:2 * HC]
    kpre = hp[:, 2 * HC:3 * HC]
    vraw_ref[...] = hp[:, 3 * HC:4 * HC]
    sh_ref[...] = ht[:, 0:128]
    ql = ht[:, 128:128 + DQ]
    ms = jnp.mean(ql * ql, axis=1, keepdims=True)
    ql = ql * lax.rsqrt(ms + 1e-6) * gq_ref[...]
    wh_ref[...] = lax.dot_general(x, ww_ref[...], dn, precision=_PREC,
                                  preferred_element_type=jnp.float32)
    cos_t = cos_ref[...]
    sin_t = sin_ref[...]
    q = lax.dot_general(ql, wuq_ref[...], dn, precision=_PREC,
                        preferred_element_type=jnp.float32)
    q_ref[...] = _rope_tiled(q, cos_t, sin_t, 64)
    qi = lax.dot_general(ql, wiuq_ref[...], dn, precision=_PREC,
                         preferred_element_type=jnp.float32)
    qi_ref[...] = _rope_tiled(qi, cosi_ref[...], sini_ref[...], 64)
    kraw_ref[...] = _rope_tiled(kpre, cos_t, sin_t, 64)


def _stage_a(x, wcat, ww, wuq, wiuq, gq, cos_t, sin_t, cosi_t, sini_t):
    T, D = x.shape
    DQ = wuq.shape[0]
    HC = 1024
    TB = 128
    grid = (T // TB,)
    wtail = wcat[:, 4 * HC:]
    wcat = wcat[:, :4 * HC]
    blk = lambda w: pl.BlockSpec((TB, w), lambda i: (i, 0))
    full = lambda a: pl.BlockSpec(a.shape, lambda i: (0, 0))
    f32 = jnp.float32
    out_shapes = [jax.ShapeDtypeStruct((T, HC), f32),
                  jax.ShapeDtypeStruct((T, HC), f32),
                  jax.ShapeDtypeStruct((T, HC), f32),
                  jax.ShapeDtypeStruct((T, HC), f32),
                  jax.ShapeDtypeStruct((T, 128), f32),
                  jax.ShapeDtypeStruct((T, HC), f32),
                  jax.ShapeDtypeStruct((T, 128), f32),
                  jax.ShapeDtypeStruct((T, 128), f32)]
    return pl.pallas_call(
        functools.partial(_stage_a_body, D=D, DQ=DQ, HC=HC),
        grid=grid,
        in_specs=[blk(D), full(wcat), full(wtail), full(ww), full(wuq),
                  full(wiuq),
                  pl.BlockSpec((1, DQ), lambda i: (0, 0)),
                  blk(HC), blk(HC), blk(128), blk(128)],
        out_specs=[blk(HC), blk(HC), blk(HC), blk(HC), blk(128), blk(HC),
                   blk(128), blk(128)],
        out_shape=out_shapes,
        compiler_params=pltpu.CompilerParams(vmem_limit_bytes=60 * 1024 * 1024),
    )(x, wcat, wtail, ww, wuq, wiuq, gq, cos_t, sin_t, cosi_t, sini_t)


def _stage_b(kv, z, sh, idx_flat, idx_T, bmh, bsh, cs4,
             P, LMAX, HC, CI):
    mesh = plsc.VectorSubcoreMesh(core_axis_name="c", subcore_axis_name="s")
    NW = 32
    PW = P // NW
    NG = PW // 16
    f32 = jnp.float32
    i32 = jnp.int32

    @functools.partial(
        pl.kernel, mesh=mesh,
        out_type=[jax.ShapeDtypeStruct((P * HC,), f32),
                  jax.ShapeDtypeStruct((P * CI,), f32),
                  jax.ShapeDtypeStruct((P,), i32),
                  jax.ShapeDtypeStruct((P, 256), f32)],
        scratch_types=[pltpu.VMEM((PW * LMAX,), i32),
                       pltpu.VMEM((LMAX * 16,), i32),
                       pltpu.VMEM((LMAX, HC), f32),
                       pltpu.VMEM((LMAX, CI), f32),
                       pltpu.VMEM((LMAX, HC), f32),
                       pltpu.VMEM((LMAX, HC), f32),
                       pltpu.VMEM((LMAX, 2 * CI), f32),
                       pltpu.VMEM((LMAX, HC), f32),
                       pltpu.VMEM((LMAX, HC), f32),
                       pltpu.VMEM((LMAX, 2 * CI), f32),
                       pltpu.VMEM((16, 256), f32),
                       pltpu.VMEM((PW * HC,), f32),
                       pltpu.VMEM((PW * CI,), f32),
                       pltpu.VMEM((PW,), i32),
                       pltpu.SemaphoreType.DMA,
                       pltpu.SemaphoreType.DMA,
                       pltpu.SemaphoreType.DMA,
                       pltpu.SemaphoreType.DMA,
                       pltpu.SemaphoreType.DMA,
                       pltpu.SemaphoreType.DMA])
    def body(kv_hbm, z_hbm, sh_hbm, idx_hbm, idxT_hbm, bmh_hbm, bsh_hbm,
             cs4_hbm,
             vph_hbm, kidx_hbm, ep_hbm, cs4pe_hbm,
             idx_v, idxT_v, bmh_v, bsh_v,
             crowsA, zrowsA, shrowsA, crowsB, zrowsB, shrowsB,
             cs4r, vph_v, kidx_v, ep_v,
             semA1, semA2, semA3, semB1, semB2, semB3):
        wid = lax.axis_index("s") * 2 + lax.axis_index("c")
        base_p = wid * PW
        pltpu.sync_copy(idx_hbm.at[pl.ds(base_p * LMAX, PW * LMAX)], idx_v)
        pltpu.sync_copy(bmh_hbm, bmh_v)
        pltpu.sync_copy(bsh_hbm, bsh_v)

        def group(g, _):
            pltpu.sync_copy(
                idxT_hbm.at[pl.ds((wid * NG + g) * 16 * LMAX, 16 * LMAX)],
                idxT_v)
            ep = idxT_v[pl.ds(0, 16)]
            for l in range(1, LMAX):
                ep = jnp.maximum(ep, idxT_v[pl.ds(l * 16, 16)])
            ep_v[pl.ds(g * 16, 16)] = ep
            pltpu.async_copy(cs4_hbm.at[ep], cs4r, semA1).wait()
            pltpu.sync_copy(cs4r, cs4pe_hbm.at[pl.ds(base_p + g * 16, 16)])
            return 0

        lax.fori_loop(0, NG, group, 0)

        bufs = ((crowsA, zrowsA, shrowsA, semA1, semA2, semA3),
                (crowsB, zrowsB, shrowsB, semB1, semB2, semB3))

        def issue(p, s):
            crows, zrows, shrows, s1, s2, s3 = bufs[s]
            idxv = idx_v[pl.ds(p * LMAX, LMAX)]
            pltpu.async_copy(kv_hbm.at[idxv], crows, s1)
            pltpu.async_copy(z_hbm.at[idxv], zrows, s2)
            pltpu.async_copy(sh_hbm.at[idxv], shrows, s3)

        def wait(s):
            crows, zrows, shrows, s1, s2, s3 = bufs[s]
            pltpu.make_async_copy(kv_hbm.at[pl.ds(0, LMAX)], crows, s1).wait()
            pltpu.make_async_copy(z_hbm.at[pl.ds(0, LMAX)], zrows, s2).wait()
            pltpu.make_async_copy(sh_hbm.at[pl.ds(0, LMAX)], shrows, s3).wait()

        def gate(p, s):
            crows, zrows, shrows = bufs[s][:3]

            def mh_chunk(c, _):
                off = c * 16
                t = [zrows[l, pl.ds(off, 16)] + bmh_v[l, pl.ds(off, 16)]
                     for l in range(LMAX)]
                m = t[0]
                for l in range(1, LMAX):
                    m = jnp.maximum(m, t[l])
                se = jnp.zeros((16,), f32)
                sec = jnp.zeros((16,), f32)
                for l in range(LMAX):
                    e = jnp.exp(t[l] - m)
                    se = se + e
                    sec = sec + e * crows[l, pl.ds(off, 16)]
                vph_v[pl.ds(p * HC + off, 16)] = sec / se
                return 0

            lax.fori_loop(0, HC // 16, mh_chunk, 0)

            def sh_chunk(c, _):
                off = c * 16
                t = [shrows[l, pl.ds(CI + off, 16)] + bsh_v[l, pl.ds(off, 16)]
                     for l in range(LMAX)]
                m = t[0]
                for l in range(1, LMAX):
                    m = jnp.maximum(m, t[l])
                se = jnp.zeros((16,), f32)
                sec = jnp.zeros((16,), f32)
                for l in range(LMAX):
                    e = jnp.exp(t[l] - m)
                    se = se + e
                    sec = sec + e * shrows[l, pl.ds(off, 16)]
                kidx_v[pl.ds(p * CI + off, 16)] = sec / se
                return 0

            lax.fori_loop(0, CI // 16, sh_chunk, 0)

        issue(0, 0)

        def pair(jj2, _):
            p0 = jj2 * 2
            issue(p0 + 1, 1)
            wait(0)
            gate(p0, 0)

            @pl.when(p0 + 2 < PW)
            def _():
                issue(p0 + 2, 0)

            wait(1)
            gate(p0 + 1, 1)
            return 0

        lax.fori_loop(0, PW // 2, pair, 0)
        pltpu.sync_copy(vph_v, vph_hbm.at[pl.ds(base_p * HC, PW * HC)])
        pltpu.sync_copy(kidx_v, kidx_hbm.at[pl.ds(base_p * CI, PW * CI)])
        pltpu.sync_copy(ep_v, ep_hbm.at[pl.ds(base_p, PW)])

    return body(kv, z, sh, idx_flat, idx_T, bmh, bsh, cs4)


def _stage_b2_body(vph_ref, kidxp_ref, cs4pe_ref, kph_ref, kidx_ref, *, H):
    cs = cs4pe_ref[...]
    cos_t = jnp.concatenate([cs[:, 0:64]] * H, axis=1)
    sin_t = jnp.concatenate([cs[:, 64:128]] * H, axis=1)
    kph_ref[...] = _rope_tiled(vph_ref[...], cos_t, sin_t, 64)
    kidx_ref[...] = _rope_tiled(kidxp_ref[...], cs[:, 128:192], cs[:, 192:256], 64)


def _stage_b2(vph, kidxp, cs4pe, H):
    P, HC = vph.shape
    CI = kidxp.shape[1]
    PB = 256
    blk = lambda w: pl.BlockSpec((PB, w), lambda i: (i, 0))
    return pl.pallas_call(
        functools.partial(_stage_b2_body, H=H),
        grid=(P // PB,),
        in_specs=[blk(HC), blk(CI), blk(256)],
        out_specs=[blk(HC), blk(CI)],
        out_shape=[jax.ShapeDtypeStruct((P, HC), jnp.float32),
                   jax.ShapeDtypeStruct((P, CI), jnp.float32)],
    )(vph, kidxp, cs4pe)


def _stage_c_body(qi_ref, kidx_ref, wh_ref, ep_ref, sel_ref, *, TB, P, TOPK):
    i = pl.program_id(0)
    qi = qi_ref[...]
    dn = (((1,), (1,)), ((), ()))
    s0 = lax.dot_general(qi[:, :64], kidx_ref[...], dn, precision=_PREC,
                         preferred_element_type=jnp.float32)
    s1 = lax.dot_general(qi[:, 64:128], kidx_ref[...], dn, precision=_PREC,
                         preferred_element_type=jnp.float32)
    wh = wh_ref[...]
    sc = jnp.maximum(s0, 0.0) * wh[:, 0:1] + jnp.maximum(s1, 0.0) * wh[:, 1:2]
    t = i * TB + lax.broadcasted_iota(jnp.int32, (TB, 1), 0)
    causal = ep_ref[...] <= t
    sc = jnp.where(causal, sc, _NEG_INF)
    iota_p = lax.broadcasted_iota(jnp.int32, (TB, P), 1)

    def it(_, carry):
        sc, sel = carry
        mval = jnp.max(sc, axis=1, keepdims=True)
        amax = jnp.min(jnp.where(sc == mval, iota_p, P), axis=1, keepdims=True)
        pick = iota_p == amax
        valid = mval > _NEG_INF
        sel = jnp.where(pick & valid, 1.0, sel)
        sc = jnp.where(pick, _NEG_INF, sc)
        return sc, sel

    _, sel = lax.fori_loop(0, TOPK, it, (sc, jnp.zeros((TB, P), jnp.float32)))
    sel_ref[...] = sel


def _stage_c(qi, kidx, wh, ep_row, TOPK):
    T = qi.shape[0]
    P = kidx.shape[0]
    TB = 128
    return pl.pallas_call(
        functools.partial(_stage_c_body, TB=TB, P=P, TOPK=TOPK),
        grid=(T // TB,),
        in_specs=[pl.BlockSpec((TB, 128), lambda i: (i, 0)),
                  pl.BlockSpec(kidx.shape, lambda i: (0, 0)),
                  pl.BlockSpec((TB, 128), lambda i: (i, 0)),
                  pl.BlockSpec((1, P), lambda i: (0, 0))],
        out_specs=pl.BlockSpec((TB, P), lambda i: (i, 0)),
        out_shape=jax.ShapeDtypeStruct((T, P), jnp.float32),
    )(qi, kidx, wh, ep_row)


def _stage_d_body(q_ref, kc_ref, kp_ref, vc_ref, vp_ref, kph_ref, vph_ref,
                  sel_ref, wo_ref, out_ref, *, TB, P, H, C):
    i = pl.program_id(0)
    scale = 1.0 / np.sqrt(np.float32(C))
    row = lax.broadcasted_iota(jnp.int32, (TB, TB), 0)
    col = lax.broadcasted_iota(jnp.int32, (TB, TB), 1)
    mask_c = col <= row
    mask_p = (row < col) & (i > 0)
    selm = sel_ref[...] > 0.0
    dnT = (((1,), (1,)), ((), ()))
    dnN = (((1,), (0,)), ((), ()))
    heads = []
    for h in range(H):
        sl = slice(h * C, (h + 1) * C)
        qh = q_ref[:, sl] * scale
        lc = lax.dot_general(qh, kc_ref[:, sl], dnT, precision=_PREC,
                             preferred_element_type=jnp.float32)
        lp = lax.dot_general(qh, kp_ref[:, sl], dnT, precision=_PREC,
                             preferred_element_type=jnp.float32)
        lph = lax.dot_general(qh, kph_ref[:, sl], dnT, precision=_PREC,
                              preferred_element_type=jnp.float32)
        lc = jnp.where(mask_c, lc, _NEG_INF)
        lp = jnp.where(mask_p, lp, _NEG_INF)
        lph = jnp.where(selm, lph, _NEG_INF)
        ec = jnp.exp(lc)
        ep = jnp.exp(lp)
        eph = jnp.exp(lph)
        den = (jnp.sum(ec, axis=1, keepdims=True)
               + jnp.sum(ep, axis=1, keepdims=True)
               + jnp.sum(eph, axis=1, keepdims=True))
        oh = (lax.dot_general(ec, vc_ref[:, sl], dnN, precision=_PREC_FAST,
                              preferred_element_type=jnp.float32)
              + lax.dot_general(ep, vp_ref[:, sl], dnN, precision=_PREC_FAST,
                                preferred_element_type=jnp.float32)
              + lax.dot_general(eph, vph_ref[:, sl], dnN, precision=_PREC_FAST,
                                preferred_element_type=jnp.float32))
        heads.append(oh / den)
    att = jnp.concatenate(heads, axis=1)
    out_ref[...] = lax.dot_general(att, wo_ref[...], dnN, precision=_PREC_FAST,
                                   preferred_element_type=jnp.float32)


def _stage_d(q, k_raw, v_raw, kph, vph, sel, wo, H, C):
    T, HC = q.shape
    P = kph.shape[0]
    TB = 128
    cur = pl.BlockSpec((TB, HC), lambda i: (i, 0))
    prev = pl.BlockSpec((TB, HC), lambda i: (jnp.maximum(i - 1, 0), 0))
    return pl.pallas_call(
        functools.partial(_stage_d_body, TB=TB, P=P, H=H, C=C),
        grid=(T // TB,),
        in_specs=[cur, cur, prev, cur, prev,
                  pl.BlockSpec(kph.shape, lambda i: (0, 0)),
                  pl.BlockSpec(vph.shape, lambda i: (0, 0)),
                  pl.BlockSpec((TB, P), lambda i: (i, 0)),
                  pl.BlockSpec(wo.shape, lambda i: (0, 0))],
        out_specs=pl.BlockSpec((TB, HC), lambda i: (i, 0)),
        out_shape=jax.ShapeDtypeStruct((T, HC), jnp.float32),
        compiler_params=pltpu.CompilerParams(vmem_limit_bytes=60 * 1024 * 1024),
    )(q, k_raw, k_raw, v_raw, v_raw, kph, vph, sel, wo)


def kernel(h, phrase_mask, phrase_token_idx, W_dq, g_q, W_uq, W_kv_mh, W_z_mh,
           B_pos_mh, W_kv_sh, W_z_sh, B_pos_sh, W_iuq, W_w, W_k, W_v, W_o):
    Bb, T, D = h.shape
    DQ = W_dq.shape[1]
    HC = W_uq.shape[1]
    C = 64
    H = HC // C
    CI = W_kv_sh.shape[1]
    P, LMAX = phrase_mask.shape[1], phrase_mask.shape[2]
    TOPK = 32

    cos_np, sin_np = _rope_cache_np(T, C)
    cosi_np, sini_np = _rope_cache_np(T, CI)
    cos_t = jnp.asarray(np.tile(cos_np, (1, H)))
    sin_t = jnp.asarray(np.tile(sin_np, (1, H)))
    cosi_t = jnp.asarray(np.tile(cosi_np, (1, 2)))
    sini_t = jnp.asarray(np.tile(sini_np, (1, 2)))
    cs4 = jnp.asarray(np.concatenate([cos_np, sin_np, cosi_np, sini_np], axis=1))

    x = h[0]
    wcat = jnp.concatenate([W_kv_mh, W_z_mh, W_k, W_v, W_kv_sh, W_z_sh, W_dq],
                           axis=1)
    ww = jnp.pad(W_w, ((0, 0), (0, 128 - W_w.shape[1])))
    gq = g_q.reshape(1, DQ)

    kv, z, k_raw, v_raw, sh, q, qi, wh = _stage_a(
        x, wcat, ww, wuq := W_uq, wiuq := W_iuq, gq, cos_t, sin_t, cosi_t, sini_t)

    idx2 = phrase_token_idx.reshape(P, LMAX).astype(jnp.int32)
    idx_flat = idx2.reshape(-1)
    idx_T = idx2.reshape(P // 16, 16, LMAX).transpose(0, 2, 1).reshape(-1)
    bmh = B_pos_mh.reshape(LMAX, HC)
    vph_f, kidxp_f, ep_f, cs4pe = _stage_b(
        kv, z, sh, idx_flat, idx_T, bmh, B_pos_sh, cs4, P, LMAX, HC, CI)
    vph = vph_f.reshape(P, HC)
    kph, kidx = _stage_b2(vph, kidxp_f.reshape(P, CI), cs4pe, H)
    ep_row = ep_f.reshape(1, P)

    sel = _stage_c(qi, kidx, wh, ep_row, TOPK)
    out = _stage_d(q, k_raw, v_raw, kph, vph, sel, W_o, H, C)
    return out.reshape(Bb, T, HC)


# speedup vs baseline: 6.7658x; 1.2524x over previous
import functools
import numpy as np
import jax
import jax.numpy as jnp
from jax import lax
from jax.experimental import pallas as pl
from jax.experimental.pallas import tpu as pltpu
from jax.experimental.pallas import tpu_sc as plsc

_PREC = lax.Precision.HIGHEST
_PREC_FAST = lax.Precision.DEFAULT
_NEG_INF = np.float32(-np.inf)


def _rope_cache_np(seq_len, head_dim, base=10000.0):
    half = head_dim // 2
    inv_freq = 1.0 / (base ** (np.arange(half, dtype=np.float64) / half))
    t = np.arange(seq_len, dtype=np.float64)
    freqs = np.outer(t, inv_freq)
    emb = np.concatenate([freqs, freqs], axis=-1)
    return np.cos(emb).astype(np.float32), np.sin(emb).astype(np.float32)


def _rope_tiled(x, c, s, hd):
    lane = lax.broadcasted_iota(jnp.int32, x.shape, 1) % hd
    lo = lane < (hd // 2)
    rot = jnp.where(lo, -jnp.roll(x, -(hd // 2), axis=1), jnp.roll(x, hd // 2, axis=1))
    return x * c + rot * s


def _stage_a_body(h_ref, wcat_ref, wtail_ref, ww_ref, wuq_ref, wiuq_ref, gq_ref,
                  cos_ref, sin_ref, cosi_ref, sini_ref,
                  kv_ref, z_ref, kraw_ref, vraw_ref, sh_ref, q_ref, qi_ref, wh_ref,
                  *, D, DQ, HC):
    x = h_ref[...]
    dn = (((1,), (0,)), ((), ()))
    hp = lax.dot_general(x, wcat_ref[...], dn, precision=_PREC_FAST,
                         preferred_element_type=jnp.float32)
    ht = lax.dot_general(x, wtail_ref[...], dn, precision=_PREC,
                         preferred_element_type=jnp.float32)
    kv_ref[...] = hp[:, 0:HC]
    z_ref[...] = hp[:, HC:2 * HC]
    kpre = hp[:, 2 * HC:3 * HC]
    vraw_ref[...] = hp[:, 3 * HC:4 * HC]
    sh_ref[...] = ht[:, 0:128]
    ql = ht[:, 128:128 + DQ]
    ms = jnp.mean(ql * ql, axis=1, keepdims=True)
    ql = ql * lax.rsqrt(ms + 1e-6) * gq_ref[...]
    wh_ref[...] = lax.dot_general(x, ww_ref[...], dn, precision=_PREC,
                                  preferred_element_type=jnp.float32)
    cos_t = cos_ref[...]
    sin_t = sin_ref[...]
    q = lax.dot_general(ql, wuq_ref[...], dn, precision=_PREC,
                        preferred_element_type=jnp.float32)
    q_ref[...] = _rope_tiled(q, cos_t, sin_t, 64)
    qi = lax.dot_general(ql, wiuq_ref[...], dn, precision=_PREC,
                         preferred_element_type=jnp.float32)
    qi_ref[...] = _rope_tiled(qi, cosi_ref[...], sini_ref[...], 64)
    kraw_ref[...] = _rope_tiled(kpre, cos_t, sin_t, 64)


def _stage_a(x, wcat, ww, wuq, wiuq, gq, cos_t, sin_t, cosi_t, sini_t):
    T, D = x.shape
    DQ = wuq.shape[0]
    HC = 1024
    TB = 128
    grid = (T // TB,)
    wtail = wcat[:, 4 * HC:]
    wcat = wcat[:, :4 * HC]
    blk = lambda w: pl.BlockSpec((TB, w), lambda i: (i, 0))
    full = lambda a: pl.BlockSpec(a.shape, lambda i: (0, 0))
    f32 = jnp.float32
    out_shapes = [jax.ShapeDtypeStruct((T, HC), f32),
                  jax.ShapeDtypeStruct((T, HC), f32),
                  jax.ShapeDtypeStruct((T, HC), f32),
                  jax.ShapeDtypeStruct((T, HC), f32),
                  jax.ShapeDtypeStruct((T, 128), f32),
                  jax.ShapeDtypeStruct((T, HC), f32),
                  jax.ShapeDtypeStruct((T, 128), f32),
                  jax.ShapeDtypeStruct((T, 128), f32)]
    return pl.pallas_call(
        functools.partial(_stage_a_body, D=D, DQ=DQ, HC=HC),
        grid=grid,
        in_specs=[blk(D), full(wcat), full(wtail), full(ww), full(wuq),
                  full(wiuq),
                  pl.BlockSpec((1, DQ), lambda i: (0, 0)),
                  blk(HC), blk(HC), blk(128), blk(128)],
        out_specs=[blk(HC), blk(HC), blk(HC), blk(HC), blk(128), blk(HC),
                   blk(128), blk(128)],
        out_shape=out_shapes,
        compiler_params=pltpu.CompilerParams(vmem_limit_bytes=60 * 1024 * 1024),
    )(x, wcat, wtail, ww, wuq, wiuq, gq, cos_t, sin_t, cosi_t, sini_t)


def _stage_b(kv, z, sh, idx_flat, idx_T, bmh, bsh, cs4,
             P, LMAX, HC, CI):
    mesh = plsc.VectorSubcoreMesh(core_axis_name="c", subcore_axis_name="s")
    NW = 32
    PW = P // NW
    NG = PW // 16
    f32 = jnp.float32
    i32 = jnp.int32

    @functools.partial(
        pl.kernel, mesh=mesh,
        out_type=[jax.ShapeDtypeStruct((P * HC,), f32),
                  jax.ShapeDtypeStruct((P * CI,), f32),
                  jax.ShapeDtypeStruct((P,), i32),
                  jax.ShapeDtypeStruct((P, 256), f32)],
        scratch_types=[pltpu.VMEM((PW * LMAX,), i32),
                       pltpu.VMEM((LMAX * 16,), i32),
                       pltpu.VMEM((LMAX, HC), f32),
                       pltpu.VMEM((LMAX, CI), f32),
                       pltpu.VMEM((LMAX, HC), f32),
                       pltpu.VMEM((LMAX, HC), f32),
                       pltpu.VMEM((LMAX, 2 * CI), f32),
                       pltpu.VMEM((LMAX, HC), f32),
                       pltpu.VMEM((LMAX, HC), f32),
                       pltpu.VMEM((LMAX, 2 * CI), f32),
                       pltpu.VMEM((16, 256), f32),
                       pltpu.VMEM((PW * HC,), f32),
                       pltpu.VMEM((PW * CI,), f32),
                       pltpu.VMEM((PW,), i32),
                       pltpu.SemaphoreType.DMA,
                       pltpu.SemaphoreType.DMA,
                       pltpu.SemaphoreType.DMA,
                       pltpu.SemaphoreType.DMA,
                       pltpu.SemaphoreType.DMA,
                       pltpu.SemaphoreType.DMA])
    def body(kv_hbm, z_hbm, sh_hbm, idx_hbm, idxT_hbm, bmh_hbm, bsh_hbm,
             cs4_hbm,
             vph_hbm, kidx_hbm, ep_hbm, cs4pe_hbm,
             idx_v, idxT_v, bmh_v, bsh_v,
             crowsA, zrowsA, shrowsA, crowsB, zrowsB, shrowsB,
             cs4r, vph_v, kidx_v, ep_v,
             semA1, semA2, semA3, semB1, semB2, semB3):
        wid = lax.axis_index("s") * 2 + lax.axis_index("c")
        base_p = wid * PW
        pltpu.sync_copy(idx_hbm.at[pl.ds(base_p * LMAX, PW * LMAX)], idx_v)
        pltpu.sync_copy(bmh_hbm, bmh_v)
        pltpu.sync_copy(bsh_hbm, bsh_v)

        def group(g, _):
            pltpu.sync_copy(
                idxT_hbm.at[pl.ds((wid * NG + g) * 16 * LMAX, 16 * LMAX)],
                idxT_v)
            ep = idxT_v[pl.ds(0, 16)]
            for l in range(1, LMAX):
                ep = jnp.maximum(ep, idxT_v[pl.ds(l * 16, 16)])
            ep_v[pl.ds(g * 16, 16)] = ep
            pltpu.async_copy(cs4_hbm.at[ep], cs4r, semA1).wait()
            pltpu.sync_copy(cs4r, cs4pe_hbm.at[pl.ds(base_p + g * 16, 16)])
            return 0

        lax.fori_loop(0, NG, group, 0)

        bufs = ((crowsA, zrowsA, shrowsA, semA1, semA2, semA3),
                (crowsB, zrowsB, shrowsB, semB1, semB2, semB3))

        def issue(p, s):
            crows, zrows, shrows, s1, s2, s3 = bufs[s]
            idxv = idx_v[pl.ds(p * LMAX, LMAX)]
            pltpu.async_copy(kv_hbm.at[idxv], crows, s1)
            pltpu.async_copy(z_hbm.at[idxv], zrows, s2)
            pltpu.async_copy(sh_hbm.at[idxv], shrows, s3)

        def wait(s):
            crows, zrows, shrows, s1, s2, s3 = bufs[s]
            pltpu.make_async_copy(kv_hbm.at[pl.ds(0, LMAX)], crows, s1).wait()
            pltpu.make_async_copy(z_hbm.at[pl.ds(0, LMAX)], zrows, s2).wait()
            pltpu.make_async_copy(sh_hbm.at[pl.ds(0, LMAX)], shrows, s3).wait()

        def gate(p, s):
            crows, zrows, shrows = bufs[s][:3]

            def mh_chunk(c, _):
                off = c * 16
                t = [zrows[l, pl.ds(off, 16)] + bmh_v[l, pl.ds(off, 16)]
                     for l in range(LMAX)]
                m = t[0]
                for l in range(1, LMAX):
                    m = jnp.maximum(m, t[l])
                se = jnp.zeros((16,), f32)
                sec = jnp.zeros((16,), f32)
                for l in range(LMAX):
                    e = jnp.exp(t[l] - m)
                    se = se + e
                    sec = sec + e * crows[l, pl.ds(off, 16)]
                vph_v[pl.ds(p * HC + off, 16)] = sec / se
                return 0

            lax.fori_loop(0, HC // 16, mh_chunk, 0)

            def sh_chunk(c, _):
                off = c * 16
                t = [shrows[l, pl.ds(CI + off, 16)] + bsh_v[l, pl.ds(off, 16)]
                     for l in range(LMAX)]
                m = t[0]
                for l in range(1, LMAX):
                    m = jnp.maximum(m, t[l])
                se = jnp.zeros((16,), f32)
                sec = jnp.zeros((16,), f32)
                for l in range(LMAX):
                    e = jnp.exp(t[l] - m)
                    se = se + e
                    sec = sec + e * shrows[l, pl.ds(off, 16)]
                kidx_v[pl.ds(p * CI + off, 16)] = sec / se
                return 0

            lax.fori_loop(0, CI // 16, sh_chunk, 0)

        issue(0, 0)

        def pair(jj2, _):
            p0 = jj2 * 2
            issue(p0 + 1, 1)
            wait(0)
            gate(p0, 0)

            @pl.when(p0 + 2 < PW)
            def _():
                issue(p0 + 2, 0)

            wait(1)
            gate(p0 + 1, 1)
            return 0

        lax.fori_loop(0, PW // 2, pair, 0)
        pltpu.sync_copy(vph_v, vph_hbm.at[pl.ds(base_p * HC, PW * HC)])
        pltpu.sync_copy(kidx_v, kidx_hbm.at[pl.ds(base_p * CI, PW * CI)])
        pltpu.sync_copy(ep_v, ep_hbm.at[pl.ds(base_p, PW)])

    return body(kv, z, sh, idx_flat, idx_T, bmh, bsh, cs4)


def _stage_b2_body(vph_ref, kidxp_ref, cs4pe_ref, kph_ref, kidx_ref, *, H):
    cs = cs4pe_ref[...]
    cos_t = jnp.concatenate([cs[:, 0:64]] * H, axis=1)
    sin_t = jnp.concatenate([cs[:, 64:128]] * H, axis=1)
    kph_ref[...] = _rope_tiled(vph_ref[...], cos_t, sin_t, 64)
    kidx_ref[...] = _rope_tiled(kidxp_ref[...], cs[:, 128:192], cs[:, 192:256], 64)


def _stage_b2(vph, kidxp, cs4pe, H):
    P, HC = vph.shape
    CI = kidxp.shape[1]
    PB = 256
    blk = lambda w: pl.BlockSpec((PB, w), lambda i: (i, 0))
    return pl.pallas_call(
        functools.partial(_stage_b2_body, H=H),
        grid=(P // PB,),
        in_specs=[blk(HC), blk(CI), blk(256)],
        out_specs=[blk(HC), blk(CI)],
        out_shape=[jax.ShapeDtypeStruct((P, HC), jnp.float32),
                   jax.ShapeDtypeStruct((P, CI), jnp.float32)],
    )(vph, kidxp, cs4pe)


def _stage_c_body(qi_ref, kidx_ref, wh_ref, ep_ref, tri_ref, sel_ref, *, TB, P, TOPK):
    i = pl.program_id(0)
    qi = qi_ref[...]
    dn = (((1,), (1,)), ((), ()))
    s0 = lax.dot_general(qi[:, :64], kidx_ref[...], dn, precision=_PREC,
                         preferred_element_type=jnp.float32)
    s1 = lax.dot_general(qi[:, 64:128], kidx_ref[...], dn, precision=_PREC,
                         preferred_element_type=jnp.float32)
    wh = wh_ref[...]
    sc = jnp.maximum(s0, 0.0) * wh[:, 0:1] + jnp.maximum(s1, 0.0) * wh[:, 1:2]
    t = i * TB + lax.broadcasted_iota(jnp.int32, (TB, 1), 0)
    causal = ep_ref[...] <= t

    si = lax.bitcast_convert_type(sc, jnp.int32)
    key = si ^ (lax.shift_right_arithmetic(si, 31) & jnp.int32(0x7FFFFFFF))
    nbig = jnp.int32(-0x80000000)
    key = jnp.where(causal, key, nbig)
    nvalid = jnp.sum(jnp.where(causal, 1.0, 0.0), axis=1, keepdims=True)
    K = jnp.minimum(nvalid, jnp.float32(TOPK))

    def cnt_ge(th):
        return jnp.sum((key >= th).astype(jnp.float32), axis=1, keepdims=True)

    zero = jnp.zeros((TB, 1), jnp.int32)
    c0 = cnt_ge(zero)
    in_pos = c0 >= K
    lo = jnp.where(in_pos, zero, jnp.full((TB, 1), nbig))
    hi = jnp.where(in_pos, jnp.full((TB, 1), jnp.int32(0x7FFFFFFF)), zero - 1)

    def it(_, carry):
        lo, hi = carry
        mid = lo + lax.shift_right_logical(hi - lo + 1, 1)
        ok = cnt_ge(mid) >= K
        return jnp.where(ok, mid, lo), jnp.where(ok, hi, mid - 1)

    theta, _ = lax.fori_loop(0, 31, it, (lo, hi))
    gt = key > theta
    cnt_gt = jnp.sum(jnp.where(gt, 1.0, 0.0), axis=1, keepdims=True)
    eq = key == theta
    rank = lax.dot_general(jnp.where(eq, 1.0, 0.0), tri_ref[...],
                           (((1,), (0,)), ((), ())), precision=_PREC_FAST,
                           preferred_element_type=jnp.float32)
    sel = gt | (eq & (rank <= K - cnt_gt))
    sel_ref[...] = jnp.where(sel, 1.0, 0.0)


def _stage_c(qi, kidx, wh, ep_row, tri, TOPK):
    T = qi.shape[0]
    P = kidx.shape[0]
    TB = 128
    return pl.pallas_call(
        functools.partial(_stage_c_body, TB=TB, P=P, TOPK=TOPK),
        grid=(T // TB,),
        in_specs=[pl.BlockSpec((TB, 128), lambda i: (i, 0)),
                  pl.BlockSpec(kidx.shape, lambda i: (0, 0)),
                  pl.BlockSpec((TB, 128), lambda i: (i, 0)),
                  pl.BlockSpec((1, P), lambda i: (0, 0)),
                  pl.BlockSpec(tri.shape, lambda i: (0, 0))],
        out_specs=pl.BlockSpec((TB, P), lambda i: (i, 0)),
        out_shape=jax.ShapeDtypeStruct((T, P), jnp.float32),
    )(qi, kidx, wh, ep_row, tri)


def _stage_d_body(q_ref, kc_ref, kp_ref, vc_ref, vp_ref, kph_ref, vph_ref,
                  sel_ref, wo_ref, out_ref, *, TB, P, H, C):
    i = pl.program_id(0)
    scale = 1.0 / np.sqrt(np.float32(C))
    row = lax.broadcasted_iota(jnp.int32, (TB, TB), 0)
    col = lax.broadcasted_iota(jnp.int32, (TB, TB), 1)
    mask_c = col <= row
    mask_p = (row < col) & (i > 0)
    selm = sel_ref[...] > 0.0
    dnT = (((1,), (1,)), ((), ()))
    dnN = (((1,), (0,)), ((), ()))
    heads = []
    for h in range(H):
        sl = slice(h * C, (h + 1) * C)
        qh = q_ref[:, sl] * scale
        lc = lax.dot_general(qh, kc_ref[:, sl], dnT, precision=_PREC,
                             preferred_element_type=jnp.float32)
        lp = lax.dot_general(qh, kp_ref[:, sl], dnT, precision=_PREC,
                             preferred_element_type=jnp.float32)
        lph = lax.dot_general(qh, kph_ref[:, sl], dnT, precision=_PREC,
                              preferred_element_type=jnp.float32)
        lc = jnp.where(mask_c, lc, _NEG_INF)
        lp = jnp.where(mask_p, lp, _NEG_INF)
        lph = jnp.where(selm, lph, _NEG_INF)
        ec = jnp.exp(lc)
        ep = jnp.exp(lp)
        eph = jnp.exp(lph)
        den = (jnp.sum(ec, axis=1, keepdims=True)
               + jnp.sum(ep, axis=1, keepdims=True)
               + jnp.sum(eph, axis=1, keepdims=True))
        oh = (lax.dot_general(ec, vc_ref[:, sl], dnN, precision=_PREC_FAST,
                              preferred_element_type=jnp.float32)
              + lax.dot_general(ep, vp_ref[:, sl], dnN, precision=_PREC_FAST,
                                preferred_element_type=jnp.float32)
              + lax.dot_general(eph, vph_ref[:, sl], dnN, precision=_PREC_FAST,
                                preferred_element_type=jnp.float32))
        heads.append(oh / den)
    att = jnp.concatenate(heads, axis=1)
    out_ref[...] = lax.dot_general(att, wo_ref[...], dnN, precision=_PREC_FAST,
                                   preferred_element_type=jnp.float32)


def _stage_d(q, k_raw, v_raw, kph, vph, sel, wo, H, C):
    T, HC = q.shape
    P = kph.shape[0]
    TB = 128
    cur = pl.BlockSpec((TB, HC), lambda i: (i, 0))
    prev = pl.BlockSpec((TB, HC), lambda i: (jnp.maximum(i - 1, 0), 0))
    return pl.pallas_call(
        functools.partial(_stage_d_body, TB=TB, P=P, H=H, C=C),
        grid=(T // TB,),
        in_specs=[cur, cur, prev, cur, prev,
                  pl.BlockSpec(kph.shape, lambda i: (0, 0)),
                  pl.BlockSpec(vph.shape, lambda i: (0, 0)),
                  pl.BlockSpec((TB, P), lambda i: (i, 0)),
                  pl.BlockSpec(wo.shape, lambda i: (0, 0))],
        out_specs=pl.BlockSpec((TB, HC), lambda i: (i, 0)),
        out_shape=jax.ShapeDtypeStruct((T, HC), jnp.float32),
        compiler_params=pltpu.CompilerParams(vmem_limit_bytes=60 * 1024 * 1024),
    )(q, k_raw, k_raw, v_raw, v_raw, kph, vph, sel, wo)


def kernel(h, phrase_mask, phrase_token_idx, W_dq, g_q, W_uq, W_kv_mh, W_z_mh,
           B_pos_mh, W_kv_sh, W_z_sh, B_pos_sh, W_iuq, W_w, W_k, W_v, W_o):
    Bb, T, D = h.shape
    DQ = W_dq.shape[1]
    HC = W_uq.shape[1]
    C = 64
    H = HC // C
    CI = W_kv_sh.shape[1]
    P, LMAX = phrase_mask.shape[1], phrase_mask.shape[2]
    TOPK = 32

    cos_np, sin_np = _rope_cache_np(T, C)
    cosi_np, sini_np = _rope_cache_np(T, CI)
    cos_t = jnp.asarray(np.tile(cos_np, (1, H)))
    sin_t = jnp.asarray(np.tile(sin_np, (1, H)))
    cosi_t = jnp.asarray(np.tile(cosi_np, (1, 2)))
    sini_t = jnp.asarray(np.tile(sini_np, (1, 2)))
    cs4 = jnp.asarray(np.concatenate([cos_np, sin_np, cosi_np, sini_np], axis=1))

    x = h[0]
    wcat = jnp.concatenate([W_kv_mh, W_z_mh, W_k, W_v, W_kv_sh, W_z_sh, W_dq],
                           axis=1)
    ww = jnp.pad(W_w, ((0, 0), (0, 128 - W_w.shape[1])))
    gq = g_q.reshape(1, DQ)

    kv, z, k_raw, v_raw, sh, q, qi, wh = _stage_a(
        x, wcat, ww, wuq := W_uq, wiuq := W_iuq, gq, cos_t, sin_t, cosi_t, sini_t)

    idx2 = phrase_token_idx.reshape(P, LMAX).astype(jnp.int32)
    idx_flat = idx2.reshape(-1)
    idx_T = idx2.reshape(P // 16, 16, LMAX).transpose(0, 2, 1).reshape(-1)
    bmh = B_pos_mh.reshape(LMAX, HC)
    vph_f, kidxp_f, ep_f, cs4pe = _stage_b(
        kv, z, sh, idx_flat, idx_T, bmh, B_pos_sh, cs4, P, LMAX, HC, CI)
    vph = vph_f.reshape(P, HC)
    kph, kidx = _stage_b2(vph, kidxp_f.reshape(P, CI), cs4pe, H)
    ep_row = ep_f.reshape(1, P)

    tri = jnp.asarray(np.triu(np.ones((P, P), np.float32)))
    sel = _stage_c(qi, kidx, wh, ep_row, tri, TOPK)
    out = _stage_d(q, k_raw, v_raw, kph, vph, sel, W_o, H, C)
    return out.reshape(Bb, T, HC)


# speedup vs baseline: 6.7660x; 1.0000x over previous
import functools
import numpy as np
import jax
import jax.numpy as jnp
from jax import lax
from jax.experimental import pallas as pl
from jax.experimental.pallas import tpu as pltpu
from jax.experimental.pallas import tpu_sc as plsc

_PREC = lax.Precision.HIGHEST
_PREC_FAST = lax.Precision.DEFAULT
_NEG_INF = np.float32(-np.inf)


def _rope_cache_np(seq_len, head_dim, base=10000.0):
    half = head_dim // 2
    inv_freq = 1.0 / (base ** (np.arange(half, dtype=np.float64) / half))
    t = np.arange(seq_len, dtype=np.float64)
    freqs = np.outer(t, inv_freq)
    emb = np.concatenate([freqs, freqs], axis=-1)
    return np.cos(emb).astype(np.float32), np.sin(emb).astype(np.float32)


def _rope_tiled(x, c, s, hd):
    lane = lax.broadcasted_iota(jnp.int32, x.shape, 1) % hd
    lo = lane < (hd // 2)
    rot = jnp.where(lo, -jnp.roll(x, -(hd // 2), axis=1), jnp.roll(x, hd // 2, axis=1))
    return x * c + rot * s


def _stage_a_body(h_ref, wcat_ref, wtail_ref, ww_ref, wuq_ref, wiuq_ref, gq_ref,
                  cos_ref, sin_ref, cosi_ref, sini_ref,
                  kv_ref, z_ref, kraw_ref, vraw_ref, sh_ref, q_ref, qi_ref, wh_ref,
                  *, D, DQ, HC):
    x = h_ref[...]
    dn = (((1,), (0,)), ((), ()))
    hp = lax.dot_general(x, wcat_ref[...], dn, precision=_PREC_FAST,
                         preferred_element_type=jnp.float32)
    ht = lax.dot_general(x, wtail_ref[...], dn, precision=_PREC,
                         preferred_element_type=jnp.float32)
    kv_ref[...] = hp[:, 0:HC]
    z_ref[...] = hp[:, HC:2 * HC]
    kpre = hp[:, 2 * HC:3 * HC]
    vraw_ref[...] = hp[:, 3 * HC:4 * HC]
    sh_ref[...] = ht[:, 0:128]
    ql = ht[:, 128:128 + DQ]
    ms = jnp.mean(ql * ql, axis=1, keepdims=True)
    ql = ql * lax.rsqrt(ms + 1e-6) * gq_ref[...]
    wh_ref[...] = lax.dot_general(x, ww_ref[...], dn, precision=_PREC,
                                  preferred_element_type=jnp.float32)
    cos_t = cos_ref[...]
    sin_t = sin_ref[...]
    q = lax.dot_general(ql, wuq_ref[...], dn, precision=_PREC,
                        preferred_element_type=jnp.float32)
    q_ref[...] = _rope_tiled(q, cos_t, sin_t, 64)
    qi = lax.dot_general(ql, wiuq_ref[...], dn, precision=_PREC,
                         preferred_element_type=jnp.float32)
    qi_ref[...] = _rope_tiled(qi, cosi_ref[...], sini_ref[...], 64)
    kraw_ref[...] = _rope_tiled(kpre, cos_t, sin_t, 64)


def _stage_a(x, wcat, ww, wuq, wiuq, gq, cos_t, sin_t, cosi_t, sini_t):
    T, D = x.shape
    DQ = wuq.shape[0]
    HC = 1024
    TB = 128
    grid = (T // TB,)
    wtail = wcat[:, 4 * HC:]
    wcat = wcat[:, :4 * HC]
    blk = lambda w: pl.BlockSpec((TB, w), lambda i: (i, 0))
    full = lambda a: pl.BlockSpec(a.shape, lambda i: (0, 0))
    f32 = jnp.float32
    out_shapes = [jax.ShapeDtypeStruct((T, HC), f32),
                  jax.ShapeDtypeStruct((T, HC), f32),
                  jax.ShapeDtypeStruct((T, HC), f32),
                  jax.ShapeDtypeStruct((T, HC), f32),
                  jax.ShapeDtypeStruct((T, 128), f32),
                  jax.ShapeDtypeStruct((T, HC), f32),
                  jax.ShapeDtypeStruct((T, 128), f32),
                  jax.ShapeDtypeStruct((T, 128), f32)]
    return pl.pallas_call(
        functools.partial(_stage_a_body, D=D, DQ=DQ, HC=HC),
        grid=grid,
        in_specs=[blk(D), full(wcat), full(wtail), full(ww), full(wuq),
                  full(wiuq),
                  pl.BlockSpec((1, DQ), lambda i: (0, 0)),
                  blk(HC), blk(HC), blk(128), blk(128)],
        out_specs=[blk(HC), blk(HC), blk(HC), blk(HC), blk(128), blk(HC),
                   blk(128), blk(128)],
        out_shape=out_shapes,
        compiler_params=pltpu.CompilerParams(vmem_limit_bytes=60 * 1024 * 1024),
    )(x, wcat, wtail, ww, wuq, wiuq, gq, cos_t, sin_t, cosi_t, sini_t)


def _stage_b(kv, z, sh, idx_flat, idx_T, bmh, bsh, cs4,
             P, LMAX, HC, CI):
    mesh = plsc.VectorSubcoreMesh(core_axis_name="c", subcore_axis_name="s")
    NW = 32
    PW = P // NW
    NG = PW // 16
    f32 = jnp.float32
    i32 = jnp.int32

    @functools.partial(
        pl.kernel, mesh=mesh,
        out_type=[jax.ShapeDtypeStruct((P * HC,), f32),
                  jax.ShapeDtypeStruct((P * CI,), f32),
                  jax.ShapeDtypeStruct((P,), i32),
                  jax.ShapeDtypeStruct((P, 256), f32)],
        scratch_types=[pltpu.VMEM((PW * LMAX,), i32),
                       pltpu.VMEM((LMAX * 16,), i32),
                       pltpu.VMEM((LMAX, HC), f32),
                       pltpu.VMEM((LMAX, CI), f32),
                       pltpu.VMEM((LMAX, HC), f32),
                       pltpu.VMEM((LMAX, HC), f32),
                       pltpu.VMEM((LMAX, 2 * CI), f32),
                       pltpu.VMEM((LMAX, HC), f32),
                       pltpu.VMEM((LMAX, HC), f32),
                       pltpu.VMEM((LMAX, 2 * CI), f32),
                       pltpu.VMEM((16, 256), f32),
                       pltpu.VMEM((PW * HC,), f32),
                       pltpu.VMEM((PW * CI,), f32),
                       pltpu.VMEM((PW,), i32),
                       pltpu.SemaphoreType.DMA,
                       pltpu.SemaphoreType.DMA,
                       pltpu.SemaphoreType.DMA,
                       pltpu.SemaphoreType.DMA,
                       pltpu.SemaphoreType.DMA,
                       pltpu.SemaphoreType.DMA])
    def body(kv_hbm, z_hbm, sh_hbm, idx_hbm, idxT_hbm, bmh_hbm, bsh_hbm,
             cs4_hbm,
             vph_hbm, kidx_hbm, ep_hbm, cs4pe_hbm,
             idx_v, idxT_v, bmh_v, bsh_v,
             crowsA, zrowsA, shrowsA, crowsB, zrowsB, shrowsB,
             cs4r, vph_v, kidx_v, ep_v,
             semA1, semA2, semA3, semB1, semB2, semB3):
        wid = lax.axis_index("s") * 2 + lax.axis_index("c")
        base_p = wid * PW
        pltpu.sync_copy(idx_hbm.at[pl.ds(base_p * LMAX, PW * LMAX)], idx_v)
        pltpu.sync_copy(bmh_hbm, bmh_v)
        pltpu.sync_copy(bsh_hbm, bsh_v)

        def group(g, _):
            pltpu.sync_copy(
                idxT_hbm.at[pl.ds((wid * NG + g) * 16 * LMAX, 16 * LMAX)],
                idxT_v)
            ep = idxT_v[pl.ds(0, 16)]
            for l in range(1, LMAX):
                ep = jnp.maximum(ep, idxT_v[pl.ds(l * 16, 16)])
            ep_v[pl.ds(g * 16, 16)] = ep
            pltpu.async_copy(cs4_hbm.at[ep], cs4r, semA1).wait()
            pltpu.sync_copy(cs4r, cs4pe_hbm.at[pl.ds(base_p + g * 16, 16)])
            return 0

        lax.fori_loop(0, NG, group, 0)

        bufs = ((crowsA, zrowsA, shrowsA, semA1, semA2, semA3),
                (crowsB, zrowsB, shrowsB, semB1, semB2, semB3))

        def issue(p, s):
            crows, zrows, shrows, s1, s2, s3 = bufs[s]
            idxv = idx_v[pl.ds(p * LMAX, LMAX)]
            pltpu.async_copy(kv_hbm.at[idxv], crows, s1)
            pltpu.async_copy(z_hbm.at[idxv], zrows, s2)
            pltpu.async_copy(sh_hbm.at[idxv], shrows, s3)

        def wait(s):
            crows, zrows, shrows, s1, s2, s3 = bufs[s]
            pltpu.make_async_copy(kv_hbm.at[pl.ds(0, LMAX)], crows, s1).wait()
            pltpu.make_async_copy(z_hbm.at[pl.ds(0, LMAX)], zrows, s2).wait()
            pltpu.make_async_copy(sh_hbm.at[pl.ds(0, LMAX)], shrows, s3).wait()

        def gate(p, s):
            crows, zrows, shrows = bufs[s][:3]

            def mh_chunk(c, _):
                off = c * 16
                t = [zrows[l, pl.ds(off, 16)] + bmh_v[l, pl.ds(off, 16)]
                     for l in range(LMAX)]
                m = t[0]
                for l in range(1, LMAX):
                    m = jnp.maximum(m, t[l])
                se = jnp.zeros((16,), f32)
                sec = jnp.zeros((16,), f32)
                for l in range(LMAX):
                    e = jnp.exp(t[l] - m)
                    se = se + e
                    sec = sec + e * crows[l, pl.ds(off, 16)]
                vph_v[pl.ds(p * HC + off, 16)] = sec / se
                return 0

            lax.fori_loop(0, HC // 16, mh_chunk, 0)

            def sh_chunk(c, _):
                off = c * 16
                t = [shrows[l, pl.ds(CI + off, 16)] + bsh_v[l, pl.ds(off, 16)]
                     for l in range(LMAX)]
                m = t[0]
                for l in range(1, LMAX):
                    m = jnp.maximum(m, t[l])
                se = jnp.zeros((16,), f32)
                sec = jnp.zeros((16,), f32)
                for l in range(LMAX):
                    e = jnp.exp(t[l] - m)
                    se = se + e
                    sec = sec + e * shrows[l, pl.ds(off, 16)]
                kidx_v[pl.ds(p * CI + off, 16)] = sec / se
                return 0

            lax.fori_loop(0, CI // 16, sh_chunk, 0)

        issue(0, 0)

        def pair(jj2, _):
            p0 = jj2 * 2
            issue(p0 + 1, 1)
            wait(0)
            gate(p0, 0)

            @pl.when(p0 + 2 < PW)
            def _():
                issue(p0 + 2, 0)

            wait(1)
            gate(p0 + 1, 1)
            return 0

        lax.fori_loop(0, PW // 2, pair, 0)
        pltpu.sync_copy(vph_v, vph_hbm.at[pl.ds(base_p * HC, PW * HC)])
        pltpu.sync_copy(kidx_v, kidx_hbm.at[pl.ds(base_p * CI, PW * CI)])
        pltpu.sync_copy(ep_v, ep_hbm.at[pl.ds(base_p, PW)])

    return body(kv, z, sh, idx_flat, idx_T, bmh, bsh, cs4)


def _stage_b2_body(vph_ref, kidxp_ref, cs4pe_ref, kph_ref, kidx_ref, *, H):
    cs = cs4pe_ref[...]
    cos_t = jnp.concatenate([cs[:, 0:64]] * H, axis=1)
    sin_t = jnp.concatenate([cs[:, 64:128]] * H, axis=1)
    kph_ref[...] = _rope_tiled(vph_ref[...], cos_t, sin_t, 64)
    kidx_ref[...] = _rope_tiled(kidxp_ref[...], cs[:, 128:192], cs[:, 192:256], 64)


def _stage_b2(vph, kidxp, cs4pe, H):
    P, HC = vph.shape
    CI = kidxp.shape[1]
    PB = 256
    blk = lambda w: pl.BlockSpec((PB, w), lambda i: (i, 0))
    return pl.pallas_call(
        functools.partial(_stage_b2_body, H=H),
        grid=(P // PB,),
        in_specs=[blk(HC), blk(CI), blk(256)],
        out_specs=[blk(HC), blk(CI)],
        out_shape=[jax.ShapeDtypeStruct((P, HC), jnp.float32),
                   jax.ShapeDtypeStruct((P, CI), jnp.float32)],
    )(vph, kidxp, cs4pe)


def _stage_c_body(qi_ref, kidx_ref, wh_ref, ep_ref, tri_ref, sel_ref, *, TB, P, TOPK):
    i = pl.program_id(0)
    qi = qi_ref[...]
    dn = (((1,), (1,)), ((), ()))
    s0 = lax.dot_general(qi[:, :64], kidx_ref[...], dn, precision=_PREC,
                         preferred_element_type=jnp.float32)
    s1 = lax.dot_general(qi[:, 64:128], kidx_ref[...], dn, precision=_PREC,
                         preferred_element_type=jnp.float32)
    wh = wh_ref[...]
    sc = jnp.maximum(s0, 0.0) * wh[:, 0:1] + jnp.maximum(s1, 0.0) * wh[:, 1:2]
    t = i * TB + lax.broadcasted_iota(jnp.int32, (TB, 1), 0)
    causal = ep_ref[...] <= t

    si = lax.bitcast_convert_type(sc, jnp.int32)
    key = si ^ (lax.shift_right_arithmetic(si, 31) & jnp.int32(0x7FFFFFFF))
    nbig = jnp.int32(-0x80000000)
    key = jnp.where(causal, key, nbig)
    nvalid = jnp.sum(jnp.where(causal, 1.0, 0.0), axis=1, keepdims=True)
    K = jnp.minimum(nvalid, jnp.float32(TOPK))

    def cnt_ge(th):
        return jnp.sum((key >= th).astype(jnp.float32), axis=1, keepdims=True)

    zero = jnp.zeros((TB, 1), jnp.int32)
    c0 = cnt_ge(zero)
    in_pos = c0 >= K
    lo = jnp.where(in_pos, zero, jnp.full((TB, 1), nbig))
    hi = jnp.where(in_pos, jnp.full((TB, 1), jnp.int32(0x7FFFFFFF)), zero - 1)

    def it(_, carry):
        lo, hi = carry
        mid = lo + lax.shift_right_logical(hi - lo + 1, 1)
        ok = cnt_ge(mid) >= K
        return jnp.where(ok, mid, lo), jnp.where(ok, hi, mid - 1)

    theta, _ = lax.fori_loop(0, 31, it, (lo, hi))
    gt = key > theta
    cnt_gt = jnp.sum(jnp.where(gt, 1.0, 0.0), axis=1, keepdims=True)
    eq = key == theta
    rank = lax.dot_general(jnp.where(eq, 1.0, 0.0), tri_ref[...],
                           (((1,), (0,)), ((), ())), precision=_PREC_FAST,
                           preferred_element_type=jnp.float32)
    sel = gt | (eq & (rank <= K - cnt_gt))
    sel_ref[...] = jnp.where(sel, 1.0, 0.0)


def _stage_c(qi, kidx, wh, ep_row, tri, TOPK):
    T = qi.shape[0]
    P = kidx.shape[0]
    TB = 128
    return pl.pallas_call(
        functools.partial(_stage_c_body, TB=TB, P=P, TOPK=TOPK),
        grid=(T // TB,),
        in_specs=[pl.BlockSpec((TB, 128), lambda i: (i, 0)),
                  pl.BlockSpec(kidx.shape, lambda i: (0, 0)),
                  pl.BlockSpec((TB, 128), lambda i: (i, 0)),
                  pl.BlockSpec((1, P), lambda i: (0, 0)),
                  pl.BlockSpec(tri.shape, lambda i: (0, 0))],
        out_specs=pl.BlockSpec((TB, P), lambda i: (i, 0)),
        out_shape=jax.ShapeDtypeStruct((T, P), jnp.float32),
    )(qi, kidx, wh, ep_row, tri)


def _stage_d_body(q_ref, kc_ref, kp_ref, vc_ref, vp_ref, kph_ref, vph_ref,
                  sel_ref, wo_ref, out_ref, *, TB, P, H, C):
    i = pl.program_id(0)
    scale = 1.0 / np.sqrt(np.float32(C))
    row = lax.broadcasted_iota(jnp.int32, (TB, TB), 0)
    col = lax.broadcasted_iota(jnp.int32, (TB, TB), 1)
    mask_c = jnp.where(col <= row, 1.0, 0.0)
    mask_p = jnp.where((row < col) & (i > 0), 1.0, 0.0)
    selm = sel_ref[...]
    dnT = (((1,), (1,)), ((), ()))
    dnN = (((1,), (0,)), ((), ()))
    heads = []
    for h in range(H):
        sl = slice(h * C, (h + 1) * C)
        qh = q_ref[:, sl] * scale
        lc = lax.dot_general(qh, kc_ref[:, sl], dnT, precision=_PREC,
                             preferred_element_type=jnp.float32)
        lp = lax.dot_general(qh, kp_ref[:, sl], dnT, precision=_PREC,
                             preferred_element_type=jnp.float32)
        lph = lax.dot_general(qh, kph_ref[:, sl], dnT, precision=_PREC,
                              preferred_element_type=jnp.float32)
        ec = jnp.exp(lc) * mask_c
        ep = jnp.exp(lp) * mask_p
        eph = jnp.exp(lph) * selm
        den = (jnp.sum(ec, axis=1, keepdims=True)
               + jnp.sum(ep, axis=1, keepdims=True)
               + jnp.sum(eph, axis=1, keepdims=True))
        oh = (lax.dot_general(ec, vc_ref[:, sl], dnN, precision=_PREC_FAST,
                              preferred_element_type=jnp.float32)
              + lax.dot_general(ep, vp_ref[:, sl], dnN, precision=_PREC_FAST,
                                preferred_element_type=jnp.float32)
              + lax.dot_general(eph, vph_ref[:, sl], dnN, precision=_PREC_FAST,
                                preferred_element_type=jnp.float32))
        heads.append(oh / den)
    att = jnp.concatenate(heads, axis=1)
    out_ref[...] = lax.dot_general(att, wo_ref[...], dnN, precision=_PREC_FAST,
                                   preferred_element_type=jnp.float32)


def _stage_d(q, k_raw, v_raw, kph, vph, sel, wo, H, C):
    T, HC = q.shape
    P = kph.shape[0]
    TB = 128
    cur = pl.BlockSpec((TB, HC), lambda i: (i, 0))
    prev = pl.BlockSpec((TB, HC), lambda i: (jnp.maximum(i - 1, 0), 0))
    return pl.pallas_call(
        functools.partial(_stage_d_body, TB=TB, P=P, H=H, C=C),
        grid=(T // TB,),
        in_specs=[cur, cur, prev, cur, prev,
                  pl.BlockSpec(kph.shape, lambda i: (0, 0)),
                  pl.BlockSpec(vph.shape, lambda i: (0, 0)),
                  pl.BlockSpec((TB, P), lambda i: (i, 0)),
                  pl.BlockSpec(wo.shape, lambda i: (0, 0))],
        out_specs=pl.BlockSpec((TB, HC), lambda i: (i, 0)),
        out_shape=jax.ShapeDtypeStruct((T, HC), jnp.float32),
        compiler_params=pltpu.CompilerParams(vmem_limit_bytes=60 * 1024 * 1024),
    )(q, k_raw, k_raw, v_raw, v_raw, kph, vph, sel, wo)


def kernel(h, phrase_mask, phrase_token_idx, W_dq, g_q, W_uq, W_kv_mh, W_z_mh,
           B_pos_mh, W_kv_sh, W_z_sh, B_pos_sh, W_iuq, W_w, W_k, W_v, W_o):
    Bb, T, D = h.shape
    DQ = W_dq.shape[1]
    HC = W_uq.shape[1]
    C = 64
    H = HC // C
    CI = W_kv_sh.shape[1]
    P, LMAX = phrase_mask.shape[1], phrase_mask.shape[2]
    TOPK = 32

    cos_np, sin_np = _rope_cache_np(T, C)
    cosi_np, sini_np = _rope_cache_np(T, CI)
    cos_t = jnp.asarray(np.tile(cos_np, (1, H)))
    sin_t = jnp.asarray(np.tile(sin_np, (1, H)))
    cosi_t = jnp.asarray(np.tile(cosi_np, (1, 2)))
    sini_t = jnp.asarray(np.tile(sini_np, (1, 2)))
    cs4 = jnp.asarray(np.concatenate([cos_np, sin_np, cosi_np, sini_np], axis=1))

    x = h[0]
    wcat = jnp.concatenate([W_kv_mh, W_z_mh, W_k, W_v, W_kv_sh, W_z_sh, W_dq],
                           axis=1)
    ww = jnp.pad(W_w, ((0, 0), (0, 128 - W_w.shape[1])))
    gq = g_q.reshape(1, DQ)

    kv, z, k_raw, v_raw, sh, q, qi, wh = _stage_a(
        x, wcat, ww, wuq := W_uq, wiuq := W_iuq, gq, cos_t, sin_t, cosi_t, sini_t)

    idx2 = phrase_token_idx.reshape(P, LMAX).astype(jnp.int32)
    idx_flat = idx2.reshape(-1)
    idx_T = idx2.reshape(P // 16, 16, LMAX).transpose(0, 2, 1).reshape(-1)
    bmh = B_pos_mh.reshape(LMAX, HC)
    vph_f, kidxp_f, ep_f, cs4pe = _stage_b(
        kv, z, sh, idx_flat, idx_T, bmh, B_pos_sh, cs4, P, LMAX, HC, CI)
    vph = vph_f.reshape(P, HC)
    kph, kidx = _stage_b2(vph, kidxp_f.reshape(P, CI), cs4pe, H)
    ep_row = ep_f.reshape(1, P)

    tri = jnp.asarray(np.triu(np.ones((P, P), np.float32)))
    sel = _stage_c(qi, kidx, wh, ep_row, tri, TOPK)
    out = _stage_d(q, k_raw, v_raw, kph, vph, sel, W_o, H, C)
    return out.reshape(Bb, T, HC)
